# baseline (speedup 1.0000x reference)
import functools

import jax
import jax.numpy as jnp
from jax import lax
from jax.experimental import pallas as pl
from jax.experimental.pallas import tpu as pltpu
from jax.experimental.pallas import tpu_sc as plsc

N_NODES = 10000
N_FEAT = 128
N_HID = 64
N_CLS = 32

NPAD = 10240
BLK = 1024
NW = 32
NC = 2
NS = 16
ROWS_PER_TILE = NPAD // NS
CHUNK = 128


def _n_chunks(e):
    ch = -(-e // (NW * CHUNK))
    return ch + (ch % 2)


def _pad_edges(idx, n_chunks):
    e = idx.shape[0]
    total = NW * n_chunks * CHUNK
    pad = jnp.full((total - e,), N_NODES, dtype=jnp.int32)
    return jnp.concatenate([idx.astype(jnp.int32), pad]).reshape(NW, n_chunks, CHUNK)


def _sc_mesh():
    return plsc.VectorSubcoreMesh(core_axis_name="c", subcore_axis_name="s",
                                  num_cores=NC, num_subcores=NS)


def _make_deg_kernel(ch_r, ch_k):
    out_t = (jax.ShapeDtypeStruct((NC, NPAD, 16), jnp.float32),
             jax.ShapeDtypeStruct((NC, NPAD, 16), jnp.float32))

    @functools.partial(
        pl.kernel,
        out_type=out_t,
        mesh=_sc_mesh(),
        compiler_params=pltpu.CompilerParams(use_tc_tiling_on_sc=False),
        scratch_types=[
            pltpu.VMEM((ch_r, CHUNK), jnp.int32),
            pltpu.VMEM((ch_k, CHUNK), jnp.int32),
            pltpu.VMEM((CHUNK, 16), jnp.float32),
            pltpu.VMEM_SHARED((NPAD, 16), jnp.float32),
            pltpu.VMEM_SHARED((NPAD, 16), jnp.float32),
        ],
    )
    def deg_kernel(dstr_hbm, dstk_hbm, ones_hbm, zeros_hbm, outr_hbm, outk_hbm,
                   dstr_v, dstk_v, ones_v, acc_r, acc_k):
        c = lax.axis_index("c")
        s = lax.axis_index("s")
        w = s * NC + c
        r0 = s * ROWS_PER_TILE
        pltpu.sync_copy(zeros_hbm.at[pl.ds(r0, ROWS_PER_TILE)],
                        acc_r.at[pl.ds(r0, ROWS_PER_TILE)])
        pltpu.sync_copy(zeros_hbm.at[pl.ds(r0, ROWS_PER_TILE)],
                        acc_k.at[pl.ds(r0, ROWS_PER_TILE)])
        pltpu.sync_copy(dstr_hbm.at[w], dstr_v)
        pltpu.sync_copy(dstk_hbm.at[w], dstk_v)
        pltpu.sync_copy(ones_hbm, ones_v)
        plsc.subcore_barrier()

        def body_r(j, carry):
            pltpu.sync_copy(ones_v, acc_r.at[dstr_v.at[j]], add=True)
            return carry

        lax.fori_loop(0, ch_r, body_r, 0)

        def body_k(j, carry):
            pltpu.sync_copy(ones_v, acc_k.at[dstk_v.at[j]], add=True)
            return carry

        lax.fori_loop(0, ch_k, body_k, 0)
        plsc.subcore_barrier()
        pltpu.sync_copy(acc_r.at[pl.ds(r0, ROWS_PER_TILE)],
                        outr_hbm.at[c].at[pl.ds(r0, ROWS_PER_TILE)])
        pltpu.sync_copy(acc_k.at[pl.ds(r0, ROWS_PER_TILE)],
                        outk_hbm.at[c].at[pl.ds(r0, ROWS_PER_TILE)])

    return deg_kernel


def _make_agg_kernel(feat, ch_r, ch_k):
    out_t = (jax.ShapeDtypeStruct((NC, NPAD, feat), jnp.float32),
             jax.ShapeDtypeStruct((NC, NPAD, feat), jnp.float32))

    @functools.partial(
        pl.kernel,
        out_type=out_t,
        mesh=_sc_mesh(),
        compiler_params=pltpu.CompilerParams(use_tc_tiling_on_sc=False),
        scratch_types=[
            pltpu.VMEM((ch_r, CHUNK), jnp.int32),
            pltpu.VMEM((ch_r, CHUNK), jnp.int32),
            pltpu.VMEM((ch_k, CHUNK), jnp.int32),
            pltpu.VMEM((ch_k, CHUNK), jnp.int32),
            pltpu.VMEM((CHUNK, feat), jnp.float32),
            pltpu.VMEM((CHUNK, feat), jnp.float32),
            pltpu.VMEM_SHARED((NPAD, feat), jnp.float32),
            pltpu.VMEM_SHARED((NPAD, feat), jnp.float32),
            pltpu.SemaphoreType.DMA,
            pltpu.SemaphoreType.DMA,
        ],
    )
    def agg_kernel(table_r, table_k, srcr_hbm, dstr_hbm, srck_hbm, dstk_hbm,
                   zeros_hbm, outr_hbm, outk_hbm,
                   srcr_v, dstr_v, srck_v, dstk_v, buf0, buf1, acc_r, acc_k,
                   sem0, sem1):
        c = lax.axis_index("c")
        s = lax.axis_index("s")
        w = s * NC + c
        r0 = s * ROWS_PER_TILE
        pltpu.sync_copy(zeros_hbm.at[pl.ds(r0, ROWS_PER_TILE)],
                        acc_r.at[pl.ds(r0, ROWS_PER_TILE)])
        pltpu.sync_copy(zeros_hbm.at[pl.ds(r0, ROWS_PER_TILE)],
                        acc_k.at[pl.ds(r0, ROWS_PER_TILE)])
        pltpu.sync_copy(srcr_hbm.at[w], srcr_v)
        pltpu.sync_copy(dstr_hbm.at[w], dstr_v)
        pltpu.sync_copy(srck_hbm.at[w], srck_v)
        pltpu.sync_copy(dstk_hbm.at[w], dstk_v)
        plsc.subcore_barrier()

        def pipelined(table, src_v, dst_v, acc, n_chunks):
            def body(j, carry):
                pltpu.async_copy(table.at[src_v.at[j]], buf0, sem0).wait()
                pltpu.sync_copy(buf0, acc.at[dst_v.at[j]], add=True)
                return carry

            lax.fori_loop(0, n_chunks, body, 0)

        pipelined(table_r, srcr_v, dstr_v, acc_r, ch_r)
        pipelined(table_k, srck_v, dstk_v, acc_k, ch_k)
        plsc.subcore_barrier()
        pltpu.sync_copy(acc_r.at[pl.ds(r0, ROWS_PER_TILE)],
                        outr_hbm.at[c].at[pl.ds(r0, ROWS_PER_TILE)])
        pltpu.sync_copy(acc_k.at[pl.ds(r0, ROWS_PER_TILE)],
                        outk_hbm.at[c].at[pl.ds(r0, ROWS_PER_TILE)])

    return agg_kernel


def _tc1(x_pad, W1, degp_r, degp_k):
    grid = NPAD // BLK

    def body(x_ref, w_ref, dr_ref, dk_ref,
             h1_ref, hsr_ref, hsk_ref, dvr_ref, dvk_ref):
        h1 = jnp.dot(x_ref[...], w_ref[...], preferred_element_type=jnp.float32)
        deg_r = dr_ref[0, :, 0:1] + dr_ref[1, :, 0:1] + 1.0
        deg_k = dk_ref[0, :, 0:1] + dk_ref[1, :, 0:1] + 1.0
        dinv_r = lax.rsqrt(deg_r)
        dinv_k = lax.rsqrt(deg_k)
        h1_ref[...] = h1
        hsr_ref[...] = h1 * dinv_r
        hsk_ref[...] = h1 * dinv_k
        dvr_ref[...] = dinv_r
        dvk_ref[...] = dinv_k

    return pl.pallas_call(
        body,
        grid=(grid,),
        in_specs=[
            pl.BlockSpec((BLK, N_FEAT), lambda i: (i, 0)),
            pl.BlockSpec((N_FEAT, N_HID), lambda i: (0, 0)),
            pl.BlockSpec((NC, BLK, 16), lambda i: (0, i, 0)),
            pl.BlockSpec((NC, BLK, 16), lambda i: (0, i, 0)),
        ],
        out_specs=[
            pl.BlockSpec((BLK, N_HID), lambda i: (i, 0)),
            pl.BlockSpec((BLK, N_HID), lambda i: (i, 0)),
            pl.BlockSpec((BLK, N_HID), lambda i: (i, 0)),
            pl.BlockSpec((BLK, 1), lambda i: (i, 0)),
            pl.BlockSpec((BLK, 1), lambda i: (i, 0)),
        ],
        out_shape=[
            jax.ShapeDtypeStruct((NPAD, N_HID), jnp.float32),
            jax.ShapeDtypeStruct((NPAD, N_HID), jnp.float32),
            jax.ShapeDtypeStruct((NPAD, N_HID), jnp.float32),
            jax.ShapeDtypeStruct((NPAD, 1), jnp.float32),
            jax.ShapeDtypeStruct((NPAD, 1), jnp.float32),
        ],
    )(x_pad, W1, degp_r, degp_k)


def _tc2(aggp_r, aggp_k, h1, dinv_r, dinv_k, b1, W2):
    grid = NPAD // BLK

    def body(ar_ref, ak_ref, h1_ref, dvr_ref, dvk_ref, b1_ref, w2_ref,
             h2_ref, hsr_ref, hsk_ref):
        dvr = dvr_ref[...]
        dvk = dvk_ref[...]
        h1 = h1_ref[...]
        b1 = b1_ref[...]
        conv_r = dvr * (ar_ref[0] + ar_ref[1]) + (dvr * dvr) * h1 + b1
        conv_k = dvk * (ak_ref[0] + ak_ref[1]) + (dvk * dvk) * h1 + b1
        r1 = jax.nn.relu(jnp.concatenate([conv_r, conv_k], axis=1))
        h2 = jnp.dot(r1, w2_ref[...], preferred_element_type=jnp.float32)
        h2_ref[...] = h2
        hsr_ref[...] = h2 * dvr
        hsk_ref[...] = h2 * dvk

    return pl.pallas_call(
        body,
        grid=(grid,),
        in_specs=[
            pl.BlockSpec((NC, BLK, N_HID), lambda i: (0, i, 0)),
            pl.BlockSpec((NC, BLK, N_HID), lambda i: (0, i, 0)),
            pl.BlockSpec((BLK, N_HID), lambda i: (i, 0)),
            pl.BlockSpec((BLK, 1), lambda i: (i, 0)),
            pl.BlockSpec((BLK, 1), lambda i: (i, 0)),
            pl.BlockSpec((1, N_HID), lambda i: (0, 0)),
            pl.BlockSpec((2 * N_HID, N_CLS), lambda i: (0, 0)),
        ],
        out_specs=[
            pl.BlockSpec((BLK, N_CLS), lambda i: (i, 0)),
            pl.BlockSpec((BLK, N_CLS), lambda i: (i, 0)),
            pl.BlockSpec((BLK, N_CLS), lambda i: (i, 0)),
        ],
        out_shape=[
            jax.ShapeDtypeStruct((NPAD, N_CLS), jnp.float32),
            jax.ShapeDtypeStruct((NPAD, N_CLS), jnp.float32),
            jax.ShapeDtypeStruct((NPAD, N_CLS), jnp.float32),
        ],
    )(aggp_r, aggp_k, h1, dinv_r, dinv_k, b1, W2)


def _tc3(aggp_r, aggp_k, h2, dinv_r, dinv_k, b2, Wl, bl):
    grid = NPAD // BLK

    def body(ar_ref, ak_ref, h2_ref, dvr_ref, dvk_ref, b2_ref, wl_ref, bl_ref,
             out_ref):
        dvr = dvr_ref[...]
        dvk = dvk_ref[...]
        h2 = h2_ref[...]
        b2 = b2_ref[...]
        conv_r = dvr * (ar_ref[0] + ar_ref[1]) + (dvr * dvr) * h2 + b2
        conv_k = dvk * (ak_ref[0] + ak_ref[1]) + (dvk * dvk) * h2 + b2
        r2 = jnp.concatenate([conv_r, conv_k], axis=1)
        final = jnp.dot(r2, wl_ref[...], preferred_element_type=jnp.float32)
        final = final + bl_ref[...]
        m = jnp.max(final, axis=1, keepdims=True)
        lse = jnp.log(jnp.sum(jnp.exp(final - m), axis=1, keepdims=True)) + m
        out_ref[...] = final - lse

    return pl.pallas_call(
        body,
        grid=(grid,),
        in_specs=[
            pl.BlockSpec((NC, BLK, N_CLS), lambda i: (0, i, 0)),
            pl.BlockSpec((NC, BLK, N_CLS), lambda i: (0, i, 0)),
            pl.BlockSpec((BLK, N_CLS), lambda i: (i, 0)),
            pl.BlockSpec((BLK, 1), lambda i: (i, 0)),
            pl.BlockSpec((BLK, 1), lambda i: (i, 0)),
            pl.BlockSpec((1, N_CLS), lambda i: (0, 0)),
            pl.BlockSpec((2 * N_CLS, N_CLS), lambda i: (0, 0)),
            pl.BlockSpec((1, N_CLS), lambda i: (0, 0)),
        ],
        out_specs=pl.BlockSpec((BLK, N_CLS), lambda i: (i, 0)),
        out_shape=jax.ShapeDtypeStruct((NPAD, N_CLS), jnp.float32),
    )(aggp_r, aggp_k, h2, dinv_r, dinv_k, b2, Wl, bl)


def kernel(x, edge_index, edge_index_knn, W1, b1, W2, b2, Wl, bl):
    e_r = edge_index.shape[1]
    e_k = edge_index_knn.shape[1]
    ch_r = _n_chunks(e_r)
    ch_k = _n_chunks(e_k)

    src_r = _pad_edges(edge_index[0], ch_r)
    dst_r = _pad_edges(edge_index[1], ch_r)
    src_k = _pad_edges(edge_index_knn[0], ch_k)
    dst_k = _pad_edges(edge_index_knn[1], ch_k)

    x_pad = jnp.zeros((NPAD, N_FEAT), jnp.float32).at[:N_NODES].set(x)
    ones16 = jnp.ones((CHUNK, 16), jnp.float32)
    zeros16 = jnp.zeros((NPAD, 16), jnp.float32)
    zeros_h = jnp.zeros((NPAD, N_HID), jnp.float32)
    zeros_c = jnp.zeros((NPAD, N_CLS), jnp.float32)
    b1r = b1.reshape(1, N_HID)
    b2r = b2.reshape(1, N_CLS)
    blr = bl.reshape(1, N_CLS)

    degp_r, degp_k = _make_deg_kernel(ch_r, ch_k)(dst_r, dst_k, ones16, zeros16)

    h1, hs1_r, hs1_k, dinv_r, dinv_k = _tc1(x_pad, W1, degp_r, degp_k)

    agg1 = _make_agg_kernel(N_HID, ch_r, ch_k)
    aggp1_r, aggp1_k = agg1(hs1_r, hs1_k, src_r, dst_r, src_k, dst_k, zeros_h)

    h2, hs2_r, hs2_k = _tc2(aggp1_r, aggp1_k, h1, dinv_r, dinv_k, b1r, W2)

    agg2 = _make_agg_kernel(N_CLS, ch_r, ch_k)
    aggp2_r, aggp2_k = agg2(hs2_r, hs2_k, src_r, dst_r, src_k, dst_k, zeros_c)

    out = _tc3(aggp2_r, aggp2_k, h2, dinv_r, dinv_k, b2r, Wl, blr)
    return out[:N_NODES]


# speedup vs baseline: 1.2819x; 1.2819x over previous
import functools

import jax
import jax.numpy as jnp
from jax import lax
from jax.experimental import pallas as pl
from jax.experimental.pallas import tpu as pltpu
from jax.experimental.pallas import tpu_sc as plsc

N_NODES = 10000
N_FEAT = 128
N_HID = 64
N_CLS = 32

NPAD = 10240
BLK = 1024
NW = 32
NC = 2
NS = 16
ROWS_PER_TILE = NPAD // NS
CHUNK = 128


def _n_chunks(e):
    ch = -(-e // (NW * CHUNK))
    return ch


def _pad_edges(idx, n_chunks):
    e = idx.shape[0]
    total = NW * n_chunks * CHUNK
    pad = jnp.full((total - e,), N_NODES, dtype=jnp.int32)
    return jnp.concatenate([idx.astype(jnp.int32), pad]).reshape(NW, n_chunks, CHUNK)


def _sc_mesh():
    return plsc.VectorSubcoreMesh(core_axis_name="c", subcore_axis_name="s",
                                  num_cores=NC, num_subcores=NS)


def _make_deg_kernel(ch_r, ch_k):
    out_t = (jax.ShapeDtypeStruct((NC, NPAD, 16), jnp.float32),
             jax.ShapeDtypeStruct((NC, NPAD, 16), jnp.float32))

    @functools.partial(
        pl.kernel,
        out_type=out_t,
        mesh=_sc_mesh(),
        compiler_params=pltpu.CompilerParams(use_tc_tiling_on_sc=False),
        scratch_types=[
            pltpu.VMEM((ch_r, CHUNK), jnp.int32),
            pltpu.VMEM((ch_k, CHUNK), jnp.int32),
            pltpu.VMEM((CHUNK, 16), jnp.float32),
            pltpu.VMEM_SHARED((NPAD, 16), jnp.float32),
            pltpu.VMEM_SHARED((NPAD, 16), jnp.float32),
        ],
    )
    def deg_kernel(dstr_hbm, dstk_hbm, ones_hbm, zeros_hbm, outr_hbm, outk_hbm,
                   dstr_v, dstk_v, ones_v, acc_r, acc_k):
        c = lax.axis_index("c")
        s = lax.axis_index("s")
        w = s * NC + c
        r0 = s * ROWS_PER_TILE
        pltpu.sync_copy(zeros_hbm.at[pl.ds(r0, ROWS_PER_TILE)],
                        acc_r.at[pl.ds(r0, ROWS_PER_TILE)])
        pltpu.sync_copy(zeros_hbm.at[pl.ds(r0, ROWS_PER_TILE)],
                        acc_k.at[pl.ds(r0, ROWS_PER_TILE)])
        pltpu.sync_copy(dstr_hbm.at[w], dstr_v)
        pltpu.sync_copy(dstk_hbm.at[w], dstk_v)
        pltpu.sync_copy(ones_hbm, ones_v)
        plsc.subcore_barrier()

        def body_r(j, carry):
            pltpu.sync_copy(ones_v, acc_r.at[dstr_v.at[j]], add=True)
            return carry

        lax.fori_loop(0, ch_r, body_r, 0)

        def body_k(j, carry):
            pltpu.sync_copy(ones_v, acc_k.at[dstk_v.at[j]], add=True)
            return carry

        lax.fori_loop(0, ch_k, body_k, 0)
        plsc.subcore_barrier()
        pltpu.sync_copy(acc_r.at[pl.ds(r0, ROWS_PER_TILE)],
                        outr_hbm.at[c].at[pl.ds(r0, ROWS_PER_TILE)])
        pltpu.sync_copy(acc_k.at[pl.ds(r0, ROWS_PER_TILE)],
                        outk_hbm.at[c].at[pl.ds(r0, ROWS_PER_TILE)])

    return deg_kernel


def _make_agg_kernel(feat, ch_r, ch_k):
    out_t = (jax.ShapeDtypeStruct((NC, NPAD, feat), jnp.float32),
             jax.ShapeDtypeStruct((NC, NPAD, feat), jnp.float32))

    @functools.partial(
        pl.kernel,
        out_type=out_t,
        mesh=_sc_mesh(),
        compiler_params=pltpu.CompilerParams(use_tc_tiling_on_sc=False),
        scratch_types=[
            pltpu.VMEM((ch_r, CHUNK), jnp.int32),
            pltpu.VMEM((ch_r, CHUNK), jnp.int32),
            pltpu.VMEM((ch_k, CHUNK), jnp.int32),
            pltpu.VMEM((ch_k, CHUNK), jnp.int32),
            pltpu.VMEM((CHUNK, feat), jnp.float32),
            pltpu.VMEM((CHUNK, feat), jnp.float32),
            pltpu.VMEM_SHARED((NPAD, feat), jnp.float32),
            pltpu.VMEM_SHARED((NPAD, feat), jnp.float32),
            pltpu.SemaphoreType.DMA,
            pltpu.SemaphoreType.DMA,
        ],
    )
    def agg_kernel(table_r, table_k, srcr_hbm, dstr_hbm, srck_hbm, dstk_hbm,
                   zeros_hbm, outr_hbm, outk_hbm,
                   srcr_v, dstr_v, srck_v, dstk_v, buf0, buf1, acc_r, acc_k,
                   sem0, sem1):
        c = lax.axis_index("c")
        s = lax.axis_index("s")
        w = s * NC + c
        r0 = s * ROWS_PER_TILE
        pltpu.sync_copy(zeros_hbm.at[pl.ds(r0, ROWS_PER_TILE)],
                        acc_r.at[pl.ds(r0, ROWS_PER_TILE)])
        pltpu.sync_copy(zeros_hbm.at[pl.ds(r0, ROWS_PER_TILE)],
                        acc_k.at[pl.ds(r0, ROWS_PER_TILE)])
        pltpu.sync_copy(srcr_hbm.at[w], srcr_v)
        pltpu.sync_copy(dstr_hbm.at[w], dstr_v)
        pltpu.sync_copy(srck_hbm.at[w], srck_v)
        pltpu.sync_copy(dstk_hbm.at[w], dstk_v)
        plsc.subcore_barrier()

        def pipelined(table, src_v, dst_v, acc, n_chunks):
            def body(j, carry):
                pltpu.async_copy(table.at[src_v.at[j]], buf0, sem0).wait()
                pltpu.sync_copy(buf0, acc.at[dst_v.at[j]], add=True)
                return carry

            lax.fori_loop(0, n_chunks, body, 0)

        pipelined(table_r, srcr_v, dstr_v, acc_r, ch_r)
        pipelined(table_k, srck_v, dstk_v, acc_k, ch_k)
        plsc.subcore_barrier()
        pltpu.sync_copy(acc_r.at[pl.ds(r0, ROWS_PER_TILE)],
                        outr_hbm.at[c].at[pl.ds(r0, ROWS_PER_TILE)])
        pltpu.sync_copy(acc_k.at[pl.ds(r0, ROWS_PER_TILE)],
                        outk_hbm.at[c].at[pl.ds(r0, ROWS_PER_TILE)])

    return agg_kernel


def _tc1(x_pad, W1, degp_r, degp_k):
    grid = NPAD // BLK

    def body(x_ref, w_ref, dr_ref, dk_ref,
             h1_ref, hsr_ref, hsk_ref, dvr_ref, dvk_ref):
        h1 = jnp.dot(x_ref[...], w_ref[...], preferred_element_type=jnp.float32)
        deg_r = dr_ref[0, :, 0:1] + dr_ref[1, :, 0:1] + 1.0
        deg_k = dk_ref[0, :, 0:1] + dk_ref[1, :, 0:1] + 1.0
        dinv_r = lax.rsqrt(deg_r)
        dinv_k = lax.rsqrt(deg_k)
        h1_ref[...] = h1
        hsr_ref[...] = h1 * dinv_r
        hsk_ref[...] = h1 * dinv_k
        dvr_ref[...] = dinv_r
        dvk_ref[...] = dinv_k

    return pl.pallas_call(
        body,
        grid=(grid,),
        in_specs=[
            pl.BlockSpec((BLK, N_FEAT), lambda i: (i, 0)),
            pl.BlockSpec((N_FEAT, N_HID), lambda i: (0, 0)),
            pl.BlockSpec((NC, BLK, 16), lambda i: (0, i, 0)),
            pl.BlockSpec((NC, BLK, 16), lambda i: (0, i, 0)),
        ],
        out_specs=[
            pl.BlockSpec((BLK, N_HID), lambda i: (i, 0)),
            pl.BlockSpec((BLK, N_HID), lambda i: (i, 0)),
            pl.BlockSpec((BLK, N_HID), lambda i: (i, 0)),
            pl.BlockSpec((BLK, 1), lambda i: (i, 0)),
            pl.BlockSpec((BLK, 1), lambda i: (i, 0)),
        ],
        out_shape=[
            jax.ShapeDtypeStruct((NPAD, N_HID), jnp.float32),
            jax.ShapeDtypeStruct((NPAD, N_HID), jnp.float32),
            jax.ShapeDtypeStruct((NPAD, N_HID), jnp.float32),
            jax.ShapeDtypeStruct((NPAD, 1), jnp.float32),
            jax.ShapeDtypeStruct((NPAD, 1), jnp.float32),
        ],
    )(x_pad, W1, degp_r, degp_k)


def _tc2(aggp_r, aggp_k, h1, dinv_r, dinv_k, b1, W2):
    grid = NPAD // BLK

    def body(ar_ref, ak_ref, h1_ref, dvr_ref, dvk_ref, b1_ref, w2_ref,
             h2_ref, hsr_ref, hsk_ref):
        dvr = dvr_ref[...]
        dvk = dvk_ref[...]
        h1 = h1_ref[...]
        b1 = b1_ref[...]
        conv_r = dvr * (ar_ref[0] + ar_ref[1]) + (dvr * dvr) * h1 + b1
        conv_k = dvk * (ak_ref[0] + ak_ref[1]) + (dvk * dvk) * h1 + b1
        r1 = jax.nn.relu(jnp.concatenate([conv_r, conv_k], axis=1))
        h2 = jnp.dot(r1, w2_ref[...], preferred_element_type=jnp.float32)
        h2_ref[...] = h2
        hsr_ref[...] = h2 * dvr
        hsk_ref[...] = h2 * dvk

    return pl.pallas_call(
        body,
        grid=(grid,),
        in_specs=[
            pl.BlockSpec((NC, BLK, N_HID), lambda i: (0, i, 0)),
            pl.BlockSpec((NC, BLK, N_HID), lambda i: (0, i, 0)),
            pl.BlockSpec((BLK, N_HID), lambda i: (i, 0)),
            pl.BlockSpec((BLK, 1), lambda i: (i, 0)),
            pl.BlockSpec((BLK, 1), lambda i: (i, 0)),
            pl.BlockSpec((1, N_HID), lambda i: (0, 0)),
            pl.BlockSpec((2 * N_HID, N_CLS), lambda i: (0, 0)),
        ],
        out_specs=[
            pl.BlockSpec((BLK, N_CLS), lambda i: (i, 0)),
            pl.BlockSpec((BLK, N_CLS), lambda i: (i, 0)),
            pl.BlockSpec((BLK, N_CLS), lambda i: (i, 0)),
        ],
        out_shape=[
            jax.ShapeDtypeStruct((NPAD, N_CLS), jnp.float32),
            jax.ShapeDtypeStruct((NPAD, N_CLS), jnp.float32),
            jax.ShapeDtypeStruct((NPAD, N_CLS), jnp.float32),
        ],
    )(aggp_r, aggp_k, h1, dinv_r, dinv_k, b1, W2)


def _tc3(aggp_r, aggp_k, h2, dinv_r, dinv_k, b2, Wl, bl):
    grid = NPAD // BLK

    def body(ar_ref, ak_ref, h2_ref, dvr_ref, dvk_ref, b2_ref, wl_ref, bl_ref,
             out_ref):
        dvr = dvr_ref[...]
        dvk = dvk_ref[...]
        h2 = h2_ref[...]
        b2 = b2_ref[...]
        conv_r = dvr * (ar_ref[0] + ar_ref[1]) + (dvr * dvr) * h2 + b2
        conv_k = dvk * (ak_ref[0] + ak_ref[1]) + (dvk * dvk) * h2 + b2
        r2 = jnp.concatenate([conv_r, conv_k], axis=1)
        final = jnp.dot(r2, wl_ref[...], preferred_element_type=jnp.float32)
        final = final + bl_ref[...]
        m = jnp.max(final, axis=1, keepdims=True)
        lse = jnp.log(jnp.sum(jnp.exp(final - m), axis=1, keepdims=True)) + m
        out_ref[...] = final - lse

    return pl.pallas_call(
        body,
        grid=(grid,),
        in_specs=[
            pl.BlockSpec((NC, BLK, N_CLS), lambda i: (0, i, 0)),
            pl.BlockSpec((NC, BLK, N_CLS), lambda i: (0, i, 0)),
            pl.BlockSpec((BLK, N_CLS), lambda i: (i, 0)),
            pl.BlockSpec((BLK, 1), lambda i: (i, 0)),
            pl.BlockSpec((BLK, 1), lambda i: (i, 0)),
            pl.BlockSpec((1, N_CLS), lambda i: (0, 0)),
            pl.BlockSpec((2 * N_CLS, N_CLS), lambda i: (0, 0)),
            pl.BlockSpec((1, N_CLS), lambda i: (0, 0)),
        ],
        out_specs=pl.BlockSpec((BLK, N_CLS), lambda i: (i, 0)),
        out_shape=jax.ShapeDtypeStruct((NPAD, N_CLS), jnp.float32),
    )(aggp_r, aggp_k, h2, dinv_r, dinv_k, b2, Wl, bl)


def kernel(x, edge_index, edge_index_knn, W1, b1, W2, b2, Wl, bl):
    e_r = edge_index.shape[1]
    e_k = edge_index_knn.shape[1]
    ch_r = _n_chunks(e_r)
    ch_k = _n_chunks(e_k)

    src_r = _pad_edges(edge_index[0], ch_r)
    dst_r = _pad_edges(edge_index[1], ch_r)
    src_k = _pad_edges(edge_index_knn[0], ch_k)
    dst_k = _pad_edges(edge_index_knn[1], ch_k)

    x_pad = jnp.zeros((NPAD, N_FEAT), jnp.float32).at[:N_NODES].set(x)
    ones16 = jnp.ones((CHUNK, 16), jnp.float32)
    zeros16 = jnp.zeros((NPAD, 16), jnp.float32)
    zeros_h = jnp.zeros((NPAD, N_HID), jnp.float32)
    zeros_c = jnp.zeros((NPAD, N_CLS), jnp.float32)
    b1r = b1.reshape(1, N_HID)
    b2r = b2.reshape(1, N_CLS)
    blr = bl.reshape(1, N_CLS)

    degp_r, degp_k = _make_deg_kernel(ch_r, ch_k)(dst_r, dst_k, ones16, zeros16)

    h1, hs1_r, hs1_k, dinv_r, dinv_k = _tc1(x_pad, W1, degp_r, degp_k)

    agg1 = _make_agg_kernel(N_HID, ch_r, ch_k)
    aggp1_r, aggp1_k = agg1(hs1_r, hs1_k, src_r, dst_r, src_k, dst_k, zeros_h)

    h2, hs2_r, hs2_k = _tc2(aggp1_r, aggp1_k, h1, dinv_r, dinv_k, b1r, W2)

    agg2 = _make_agg_kernel(N_CLS, ch_r, ch_k)
    aggp2_r, aggp2_k = agg2(hs2_r, hs2_k, src_r, dst_r, src_k, dst_k, zeros_c)

    out = _tc3(aggp2_r, aggp2_k, h2, dinv_r, dinv_k, b2r, Wl, blr)
    return out[:N_NODES]


# speedup vs baseline: 1.7344x; 1.3530x over previous
import functools

import jax
import jax.numpy as jnp
from jax import lax
from jax.experimental import pallas as pl
from jax.experimental.pallas import tpu as pltpu
from jax.experimental.pallas import tpu_sc as plsc

N_NODES = 10000
N_FEAT = 128
N_HID = 64
N_CLS = 32

NPAD = 10240
BLK = 1024
NW = 32
NC = 2
NS = 16
ROWS_PER_TILE = NPAD // NS
CHUNK = 128


def _n_chunks(e):
    ch = -(-e // (NW * CHUNK))
    return ch


def _pad_edges(idx, n_chunks):
    e = idx.shape[0]
    total = NW * n_chunks * CHUNK
    pad = N_NODES + (jnp.arange(total - e, dtype=jnp.int32) % (NPAD - N_NODES))
    return jnp.concatenate([idx.astype(jnp.int32), pad]).reshape(NW, n_chunks, CHUNK)


def _sc_mesh():
    return plsc.VectorSubcoreMesh(core_axis_name="c", subcore_axis_name="s",
                                  num_cores=NC, num_subcores=NS)


def _make_deg_kernel(ch_r, ch_k):
    out_t = (jax.ShapeDtypeStruct((NC, NPAD, 16), jnp.float32),
             jax.ShapeDtypeStruct((NC, NPAD, 16), jnp.float32))

    @functools.partial(
        pl.kernel,
        out_type=out_t,
        mesh=_sc_mesh(),
        compiler_params=pltpu.CompilerParams(use_tc_tiling_on_sc=False),
        scratch_types=[
            pltpu.VMEM((ch_r, CHUNK), jnp.int32),
            pltpu.VMEM((ch_k, CHUNK), jnp.int32),
            pltpu.VMEM((CHUNK, 16), jnp.float32),
            pltpu.VMEM_SHARED((NPAD, 16), jnp.float32),
            pltpu.VMEM_SHARED((NPAD, 16), jnp.float32),
        ],
    )
    def deg_kernel(dstr_hbm, dstk_hbm, ones_hbm, zeros_hbm, outr_hbm, outk_hbm,
                   dstr_v, dstk_v, ones_v, acc_r, acc_k):
        c = lax.axis_index("c")
        s = lax.axis_index("s")
        w = s * NC + c
        r0 = s * ROWS_PER_TILE
        pltpu.sync_copy(zeros_hbm.at[pl.ds(r0, ROWS_PER_TILE)],
                        acc_r.at[pl.ds(r0, ROWS_PER_TILE)])
        pltpu.sync_copy(zeros_hbm.at[pl.ds(r0, ROWS_PER_TILE)],
                        acc_k.at[pl.ds(r0, ROWS_PER_TILE)])
        pltpu.sync_copy(dstr_hbm.at[w], dstr_v)
        pltpu.sync_copy(dstk_hbm.at[w], dstk_v)
        pltpu.sync_copy(ones_hbm, ones_v)
        plsc.subcore_barrier()

        def body_r(j, carry):
            pltpu.sync_copy(ones_v, acc_r.at[dstr_v.at[j]], add=True)
            return carry

        lax.fori_loop(0, ch_r, body_r, 0)

        def body_k(j, carry):
            pltpu.sync_copy(ones_v, acc_k.at[dstk_v.at[j]], add=True)
            return carry

        lax.fori_loop(0, ch_k, body_k, 0)
        plsc.subcore_barrier()
        pltpu.sync_copy(acc_r.at[pl.ds(r0, ROWS_PER_TILE)],
                        outr_hbm.at[c].at[pl.ds(r0, ROWS_PER_TILE)])
        pltpu.sync_copy(acc_k.at[pl.ds(r0, ROWS_PER_TILE)],
                        outk_hbm.at[c].at[pl.ds(r0, ROWS_PER_TILE)])

    return deg_kernel


def _make_agg_kernel(feat, ch_r, ch_k):
    out_t = (jax.ShapeDtypeStruct((NC, NPAD, feat), jnp.float32),
             jax.ShapeDtypeStruct((NC, NPAD, feat), jnp.float32))

    @functools.partial(
        pl.kernel,
        out_type=out_t,
        mesh=_sc_mesh(),
        compiler_params=pltpu.CompilerParams(use_tc_tiling_on_sc=False),
        scratch_types=[
            pltpu.VMEM((ch_r, CHUNK), jnp.int32),
            pltpu.VMEM((ch_r, CHUNK), jnp.int32),
            pltpu.VMEM((ch_k, CHUNK), jnp.int32),
            pltpu.VMEM((ch_k, CHUNK), jnp.int32),
            pltpu.VMEM((CHUNK, feat), jnp.float32),
            pltpu.VMEM((CHUNK, feat), jnp.float32),
            pltpu.VMEM_SHARED((NPAD, feat), jnp.float32),
            pltpu.VMEM_SHARED((NPAD, feat), jnp.float32),
            pltpu.SemaphoreType.DMA,
            pltpu.SemaphoreType.DMA,
        ],
    )
    def agg_kernel(table_r, table_k, srcr_hbm, dstr_hbm, srck_hbm, dstk_hbm,
                   zeros_hbm, outr_hbm, outk_hbm,
                   srcr_v, dstr_v, srck_v, dstk_v, buf0, buf1, acc_r, acc_k,
                   sem0, sem1):
        c = lax.axis_index("c")
        s = lax.axis_index("s")
        w = s * NC + c
        r0 = s * ROWS_PER_TILE
        pltpu.sync_copy(zeros_hbm.at[pl.ds(r0, ROWS_PER_TILE)],
                        acc_r.at[pl.ds(r0, ROWS_PER_TILE)])
        pltpu.sync_copy(zeros_hbm.at[pl.ds(r0, ROWS_PER_TILE)],
                        acc_k.at[pl.ds(r0, ROWS_PER_TILE)])
        pltpu.sync_copy(srcr_hbm.at[w], srcr_v)
        pltpu.sync_copy(dstr_hbm.at[w], dstr_v)
        pltpu.sync_copy(srck_hbm.at[w], srck_v)
        pltpu.sync_copy(dstk_hbm.at[w], dstk_v)
        plsc.subcore_barrier()

        def pipelined(table, src_v, dst_v, acc, n_chunks):
            def body(j, carry):
                pltpu.async_copy(table.at[src_v.at[j]], buf0, sem0).wait()
                pltpu.sync_copy(buf0, acc.at[dst_v.at[j]], add=True)
                return carry

            lax.fori_loop(0, n_chunks, body, 0)

        pipelined(table_r, srcr_v, dstr_v, acc_r, ch_r)
        pipelined(table_k, srck_v, dstk_v, acc_k, ch_k)
        plsc.subcore_barrier()
        pltpu.sync_copy(acc_r.at[pl.ds(r0, ROWS_PER_TILE)],
                        outr_hbm.at[c].at[pl.ds(r0, ROWS_PER_TILE)])
        pltpu.sync_copy(acc_k.at[pl.ds(r0, ROWS_PER_TILE)],
                        outk_hbm.at[c].at[pl.ds(r0, ROWS_PER_TILE)])

    return agg_kernel


def _tc1(x_pad, W1, degp_r, degp_k):
    grid = NPAD // BLK

    def body(x_ref, w_ref, dr_ref, dk_ref,
             h1_ref, hsr_ref, hsk_ref, dvr_ref, dvk_ref):
        h1 = jnp.dot(x_ref[...], w_ref[...], preferred_element_type=jnp.float32)
        deg_r = dr_ref[0, :, 0:1] + dr_ref[1, :, 0:1] + 1.0
        deg_k = dk_ref[0, :, 0:1] + dk_ref[1, :, 0:1] + 1.0
        dinv_r = lax.rsqrt(deg_r)
        dinv_k = lax.rsqrt(deg_k)
        h1_ref[...] = h1
        hsr_ref[...] = h1 * dinv_r
        hsk_ref[...] = h1 * dinv_k
        dvr_ref[...] = dinv_r
        dvk_ref[...] = dinv_k

    return pl.pallas_call(
        body,
        grid=(grid,),
        in_specs=[
            pl.BlockSpec((BLK, N_FEAT), lambda i: (i, 0)),
            pl.BlockSpec((N_FEAT, N_HID), lambda i: (0, 0)),
            pl.BlockSpec((NC, BLK, 16), lambda i: (0, i, 0)),
            pl.BlockSpec((NC, BLK, 16), lambda i: (0, i, 0)),
        ],
        out_specs=[
            pl.BlockSpec((BLK, N_HID), lambda i: (i, 0)),
            pl.BlockSpec((BLK, N_HID), lambda i: (i, 0)),
            pl.BlockSpec((BLK, N_HID), lambda i: (i, 0)),
            pl.BlockSpec((BLK, 1), lambda i: (i, 0)),
            pl.BlockSpec((BLK, 1), lambda i: (i, 0)),
        ],
        out_shape=[
            jax.ShapeDtypeStruct((NPAD, N_HID), jnp.float32),
            jax.ShapeDtypeStruct((NPAD, N_HID), jnp.float32),
            jax.ShapeDtypeStruct((NPAD, N_HID), jnp.float32),
            jax.ShapeDtypeStruct((NPAD, 1), jnp.float32),
            jax.ShapeDtypeStruct((NPAD, 1), jnp.float32),
        ],
    )(x_pad, W1, degp_r, degp_k)


def _tc2(aggp_r, aggp_k, h1, dinv_r, dinv_k, b1, W2):
    grid = NPAD // BLK

    def body(ar_ref, ak_ref, h1_ref, dvr_ref, dvk_ref, b1_ref, w2_ref,
             h2_ref, hsr_ref, hsk_ref):
        dvr = dvr_ref[...]
        dvk = dvk_ref[...]
        h1 = h1_ref[...]
        b1 = b1_ref[...]
        conv_r = dvr * (ar_ref[0] + ar_ref[1]) + (dvr * dvr) * h1 + b1
        conv_k = dvk * (ak_ref[0] + ak_ref[1]) + (dvk * dvk) * h1 + b1
        r1 = jax.nn.relu(jnp.concatenate([conv_r, conv_k], axis=1))
        h2 = jnp.dot(r1, w2_ref[...], preferred_element_type=jnp.float32)
        h2_ref[...] = h2
        hsr_ref[...] = h2 * dvr
        hsk_ref[...] = h2 * dvk

    return pl.pallas_call(
        body,
        grid=(grid,),
        in_specs=[
            pl.BlockSpec((NC, BLK, N_HID), lambda i: (0, i, 0)),
            pl.BlockSpec((NC, BLK, N_HID), lambda i: (0, i, 0)),
            pl.BlockSpec((BLK, N_HID), lambda i: (i, 0)),
            pl.BlockSpec((BLK, 1), lambda i: (i, 0)),
            pl.BlockSpec((BLK, 1), lambda i: (i, 0)),
            pl.BlockSpec((1, N_HID), lambda i: (0, 0)),
            pl.BlockSpec((2 * N_HID, N_CLS), lambda i: (0, 0)),
        ],
        out_specs=[
            pl.BlockSpec((BLK, N_CLS), lambda i: (i, 0)),
            pl.BlockSpec((BLK, N_CLS), lambda i: (i, 0)),
            pl.BlockSpec((BLK, N_CLS), lambda i: (i, 0)),
        ],
        out_shape=[
            jax.ShapeDtypeStruct((NPAD, N_CLS), jnp.float32),
            jax.ShapeDtypeStruct((NPAD, N_CLS), jnp.float32),
            jax.ShapeDtypeStruct((NPAD, N_CLS), jnp.float32),
        ],
    )(aggp_r, aggp_k, h1, dinv_r, dinv_k, b1, W2)


def _tc3(aggp_r, aggp_k, h2, dinv_r, dinv_k, b2, Wl, bl):
    grid = NPAD // BLK

    def body(ar_ref, ak_ref, h2_ref, dvr_ref, dvk_ref, b2_ref, wl_ref, bl_ref,
             out_ref):
        dvr = dvr_ref[...]
        dvk = dvk_ref[...]
        h2 = h2_ref[...]
        b2 = b2_ref[...]
        conv_r = dvr * (ar_ref[0] + ar_ref[1]) + (dvr * dvr) * h2 + b2
        conv_k = dvk * (ak_ref[0] + ak_ref[1]) + (dvk * dvk) * h2 + b2
        r2 = jnp.concatenate([conv_r, conv_k], axis=1)
        final = jnp.dot(r2, wl_ref[...], preferred_element_type=jnp.float32)
        final = final + bl_ref[...]
        m = jnp.max(final, axis=1, keepdims=True)
        lse = jnp.log(jnp.sum(jnp.exp(final - m), axis=1, keepdims=True)) + m
        out_ref[...] = final - lse

    return pl.pallas_call(
        body,
        grid=(grid,),
        in_specs=[
            pl.BlockSpec((NC, BLK, N_CLS), lambda i: (0, i, 0)),
            pl.BlockSpec((NC, BLK, N_CLS), lambda i: (0, i, 0)),
            pl.BlockSpec((BLK, N_CLS), lambda i: (i, 0)),
            pl.BlockSpec((BLK, 1), lambda i: (i, 0)),
            pl.BlockSpec((BLK, 1), lambda i: (i, 0)),
            pl.BlockSpec((1, N_CLS), lambda i: (0, 0)),
            pl.BlockSpec((2 * N_CLS, N_CLS), lambda i: (0, 0)),
            pl.BlockSpec((1, N_CLS), lambda i: (0, 0)),
        ],
        out_specs=pl.BlockSpec((BLK, N_CLS), lambda i: (i, 0)),
        out_shape=jax.ShapeDtypeStruct((NPAD, N_CLS), jnp.float32),
    )(aggp_r, aggp_k, h2, dinv_r, dinv_k, b2, Wl, bl)


def kernel(x, edge_index, edge_index_knn, W1, b1, W2, b2, Wl, bl):
    e_r = edge_index.shape[1]
    e_k = edge_index_knn.shape[1]
    ch_r = _n_chunks(e_r)
    ch_k = _n_chunks(e_k)

    src_r = _pad_edges(edge_index[0], ch_r)
    dst_r = _pad_edges(edge_index[1], ch_r)
    src_k = _pad_edges(edge_index_knn[0], ch_k)
    dst_k = _pad_edges(edge_index_knn[1], ch_k)

    x_pad = jnp.zeros((NPAD, N_FEAT), jnp.float32).at[:N_NODES].set(x)
    ones16 = jnp.ones((CHUNK, 16), jnp.float32)
    zeros16 = jnp.zeros((NPAD, 16), jnp.float32)
    zeros_h = jnp.zeros((NPAD, N_HID), jnp.float32)
    zeros_c = jnp.zeros((NPAD, N_CLS), jnp.float32)
    b1r = b1.reshape(1, N_HID)
    b2r = b2.reshape(1, N_CLS)
    blr = bl.reshape(1, N_CLS)

    degp_r, degp_k = _make_deg_kernel(ch_r, ch_k)(dst_r, dst_k, ones16, zeros16)

    h1, hs1_r, hs1_k, dinv_r, dinv_k = _tc1(x_pad, W1, degp_r, degp_k)

    agg1 = _make_agg_kernel(N_HID, ch_r, ch_k)
    aggp1_r, aggp1_k = agg1(hs1_r, hs1_k, src_r, dst_r, src_k, dst_k, zeros_h)

    h2, hs2_r, hs2_k = _tc2(aggp1_r, aggp1_k, h1, dinv_r, dinv_k, b1r, W2)

    agg2 = _make_agg_kernel(N_CLS, ch_r, ch_k)
    aggp2_r, aggp2_k = agg2(hs2_r, hs2_k, src_r, dst_r, src_k, dst_k, zeros_c)

    out = _tc3(aggp2_r, aggp2_k, h2, dinv_r, dinv_k, b2r, Wl, blr)
    return out[:N_NODES]


# speedup vs baseline: 1.9377x; 1.1172x over previous
import functools

import jax
import jax.numpy as jnp
from jax import lax
from jax.experimental import pallas as pl
from jax.experimental.pallas import tpu as pltpu
from jax.experimental.pallas import tpu_sc as plsc

N_NODES = 10000
N_FEAT = 128
N_HID = 64
N_CLS = 32

NPAD = 10240
BLK = 1024
NW = 32
NC = 2
NS = 16
ROWS_PER_TILE = NPAD // NS
CHUNK = 128


def _n_chunks(e):
    ch = -(-e // (NW * CHUNK))
    return ch + (ch % 2)


def _pad_edges(idx, n_chunks):
    e = idx.shape[0]
    total = NW * n_chunks * CHUNK
    pad = N_NODES + (jnp.arange(total - e, dtype=jnp.int32) % (NPAD - N_NODES))
    return jnp.concatenate([idx.astype(jnp.int32), pad]).reshape(NW, n_chunks, CHUNK)


def _sc_mesh():
    return plsc.VectorSubcoreMesh(core_axis_name="c", subcore_axis_name="s",
                                  num_cores=NC, num_subcores=NS)


def _make_deg_kernel(ch_r, ch_k):
    out_t = (jax.ShapeDtypeStruct((NC, NPAD, 16), jnp.float32),
             jax.ShapeDtypeStruct((NC, NPAD, 16), jnp.float32))

    @functools.partial(
        pl.kernel,
        out_type=out_t,
        mesh=_sc_mesh(),
        compiler_params=pltpu.CompilerParams(use_tc_tiling_on_sc=False),
        scratch_types=[
            pltpu.VMEM((ch_r, CHUNK), jnp.int32),
            pltpu.VMEM((ch_k, CHUNK), jnp.int32),
            pltpu.VMEM((CHUNK, 16), jnp.float32),
            pltpu.VMEM_SHARED((NPAD, 16), jnp.float32),
            pltpu.VMEM_SHARED((NPAD, 16), jnp.float32),
        ],
    )
    def deg_kernel(dstr_hbm, dstk_hbm, ones_hbm, zeros_hbm, outr_hbm, outk_hbm,
                   dstr_v, dstk_v, ones_v, acc_r, acc_k):
        c = lax.axis_index("c")
        s = lax.axis_index("s")
        w = s * NC + c
        r0 = s * ROWS_PER_TILE
        pltpu.sync_copy(zeros_hbm.at[pl.ds(r0, ROWS_PER_TILE)],
                        acc_r.at[pl.ds(r0, ROWS_PER_TILE)])
        pltpu.sync_copy(zeros_hbm.at[pl.ds(r0, ROWS_PER_TILE)],
                        acc_k.at[pl.ds(r0, ROWS_PER_TILE)])
        pltpu.sync_copy(dstr_hbm.at[w], dstr_v)
        pltpu.sync_copy(dstk_hbm.at[w], dstk_v)
        pltpu.sync_copy(ones_hbm, ones_v)
        plsc.subcore_barrier()

        def body_r(j, carry):
            pltpu.sync_copy(ones_v, acc_r.at[dstr_v.at[j]], add=True)
            return carry

        lax.fori_loop(0, ch_r, body_r, 0)

        def body_k(j, carry):
            pltpu.sync_copy(ones_v, acc_k.at[dstk_v.at[j]], add=True)
            return carry

        lax.fori_loop(0, ch_k, body_k, 0)
        plsc.subcore_barrier()
        pltpu.sync_copy(acc_r.at[pl.ds(r0, ROWS_PER_TILE)],
                        outr_hbm.at[c].at[pl.ds(r0, ROWS_PER_TILE)])
        pltpu.sync_copy(acc_k.at[pl.ds(r0, ROWS_PER_TILE)],
                        outk_hbm.at[c].at[pl.ds(r0, ROWS_PER_TILE)])

    return deg_kernel


def _make_agg_kernel(feat, ch_r, ch_k):
    out_t = (jax.ShapeDtypeStruct((NC, NPAD, feat), jnp.float32),
             jax.ShapeDtypeStruct((NC, NPAD, feat), jnp.float32))

    @functools.partial(
        pl.kernel,
        out_type=out_t,
        mesh=_sc_mesh(),
        compiler_params=pltpu.CompilerParams(use_tc_tiling_on_sc=False),
        scratch_types=[
            pltpu.VMEM((ch_r, CHUNK), jnp.int32),
            pltpu.VMEM((ch_r, CHUNK), jnp.int32),
            pltpu.VMEM((ch_k, CHUNK), jnp.int32),
            pltpu.VMEM((ch_k, CHUNK), jnp.int32),
            pltpu.VMEM((CHUNK, feat), jnp.float32),
            pltpu.VMEM((CHUNK, feat), jnp.float32),
            pltpu.VMEM_SHARED((NPAD, feat), jnp.float32),
            pltpu.VMEM_SHARED((NPAD, feat), jnp.float32),
            pltpu.SemaphoreType.DMA,
            pltpu.SemaphoreType.DMA,
        ],
    )
    def agg_kernel(table_r, table_k, srcr_hbm, dstr_hbm, srck_hbm, dstk_hbm,
                   zeros_hbm, outr_hbm, outk_hbm,
                   srcr_v, dstr_v, srck_v, dstk_v, buf0, buf1, acc_r, acc_k,
                   sem0, sem1):
        c = lax.axis_index("c")
        s = lax.axis_index("s")
        w = s * NC + c
        r0 = s * ROWS_PER_TILE
        pltpu.sync_copy(zeros_hbm.at[pl.ds(r0, ROWS_PER_TILE)],
                        acc_r.at[pl.ds(r0, ROWS_PER_TILE)])
        pltpu.sync_copy(zeros_hbm.at[pl.ds(r0, ROWS_PER_TILE)],
                        acc_k.at[pl.ds(r0, ROWS_PER_TILE)])
        pltpu.sync_copy(srcr_hbm.at[w], srcr_v)
        pltpu.sync_copy(dstr_hbm.at[w], dstr_v)
        pltpu.sync_copy(srck_hbm.at[w], srck_v)
        pltpu.sync_copy(dstk_hbm.at[w], dstk_v)
        plsc.subcore_barrier()

        def pipelined(table, src_v, dst_v, acc, n_pairs):
            pltpu.async_copy(table.at[src_v.at[0]], buf0, sem0)

            def body(i, carry):
                pltpu.make_async_copy(table.at[src_v.at[2 * i]], buf0,
                                      sem0).wait()
                pltpu.async_copy(table.at[src_v.at[2 * i + 1]], buf1, sem1)
                pltpu.sync_copy(buf0, acc.at[dst_v.at[2 * i]], add=True)
                pltpu.make_async_copy(table.at[src_v.at[2 * i + 1]], buf1,
                                      sem1).wait()

                @pl.when(i + 1 < n_pairs)
                def _():
                    pltpu.async_copy(table.at[src_v.at[2 * i + 2]], buf0, sem0)

                pltpu.sync_copy(buf1, acc.at[dst_v.at[2 * i + 1]], add=True)
                return carry

            lax.fori_loop(0, n_pairs, body, 0)

        pipelined(table_r, srcr_v, dstr_v, acc_r, ch_r // 2)
        pipelined(table_k, srck_v, dstk_v, acc_k, ch_k // 2)
        plsc.subcore_barrier()
        pltpu.sync_copy(acc_r.at[pl.ds(r0, ROWS_PER_TILE)],
                        outr_hbm.at[c].at[pl.ds(r0, ROWS_PER_TILE)])
        pltpu.sync_copy(acc_k.at[pl.ds(r0, ROWS_PER_TILE)],
                        outk_hbm.at[c].at[pl.ds(r0, ROWS_PER_TILE)])

    return agg_kernel


def _tc1(x_pad, W1, degp_r, degp_k):
    grid = NPAD // BLK

    def body(x_ref, w_ref, dr_ref, dk_ref,
             h1_ref, hsr_ref, hsk_ref, dvr_ref, dvk_ref):
        h1 = jnp.dot(x_ref[...], w_ref[...], preferred_element_type=jnp.float32)
        deg_r = dr_ref[0, :, 0:1] + dr_ref[1, :, 0:1] + 1.0
        deg_k = dk_ref[0, :, 0:1] + dk_ref[1, :, 0:1] + 1.0
        dinv_r = lax.rsqrt(deg_r)
        dinv_k = lax.rsqrt(deg_k)
        h1_ref[...] = h1
        hsr_ref[...] = h1 * dinv_r
        hsk_ref[...] = h1 * dinv_k
        dvr_ref[...] = dinv_r
        dvk_ref[...] = dinv_k

    return pl.pallas_call(
        body,
        grid=(grid,),
        in_specs=[
            pl.BlockSpec((BLK, N_FEAT), lambda i: (i, 0)),
            pl.BlockSpec((N_FEAT, N_HID), lambda i: (0, 0)),
            pl.BlockSpec((NC, BLK, 16), lambda i: (0, i, 0)),
            pl.BlockSpec((NC, BLK, 16), lambda i: (0, i, 0)),
        ],
        out_specs=[
            pl.BlockSpec((BLK, N_HID), lambda i: (i, 0)),
            pl.BlockSpec((BLK, N_HID), lambda i: (i, 0)),
            pl.BlockSpec((BLK, N_HID), lambda i: (i, 0)),
            pl.BlockSpec((BLK, 1), lambda i: (i, 0)),
            pl.BlockSpec((BLK, 1), lambda i: (i, 0)),
        ],
        out_shape=[
            jax.ShapeDtypeStruct((NPAD, N_HID), jnp.float32),
            jax.ShapeDtypeStruct((NPAD, N_HID), jnp.float32),
            jax.ShapeDtypeStruct((NPAD, N_HID), jnp.float32),
            jax.ShapeDtypeStruct((NPAD, 1), jnp.float32),
            jax.ShapeDtypeStruct((NPAD, 1), jnp.float32),
        ],
    )(x_pad, W1, degp_r, degp_k)


def _tc2(aggp_r, aggp_k, h1, dinv_r, dinv_k, b1, W2):
    grid = NPAD // BLK

    def body(ar_ref, ak_ref, h1_ref, dvr_ref, dvk_ref, b1_ref, w2_ref,
             h2_ref, hsr_ref, hsk_ref):
        dvr = dvr_ref[...]
        dvk = dvk_ref[...]
        h1 = h1_ref[...]
        b1 = b1_ref[...]
        conv_r = dvr * (ar_ref[0] + ar_ref[1]) + (dvr * dvr) * h1 + b1
        conv_k = dvk * (ak_ref[0] + ak_ref[1]) + (dvk * dvk) * h1 + b1
        r1 = jax.nn.relu(jnp.concatenate([conv_r, conv_k], axis=1))
        h2 = jnp.dot(r1, w2_ref[...], preferred_element_type=jnp.float32)
        h2_ref[...] = h2
        hsr_ref[...] = h2 * dvr
        hsk_ref[...] = h2 * dvk

    return pl.pallas_call(
        body,
        grid=(grid,),
        in_specs=[
            pl.BlockSpec((NC, BLK, N_HID), lambda i: (0, i, 0)),
            pl.BlockSpec((NC, BLK, N_HID), lambda i: (0, i, 0)),
            pl.BlockSpec((BLK, N_HID), lambda i: (i, 0)),
            pl.BlockSpec((BLK, 1), lambda i: (i, 0)),
            pl.BlockSpec((BLK, 1), lambda i: (i, 0)),
            pl.BlockSpec((1, N_HID), lambda i: (0, 0)),
            pl.BlockSpec((2 * N_HID, N_CLS), lambda i: (0, 0)),
        ],
        out_specs=[
            pl.BlockSpec((BLK, N_CLS), lambda i: (i, 0)),
            pl.BlockSpec((BLK, N_CLS), lambda i: (i, 0)),
            pl.BlockSpec((BLK, N_CLS), lambda i: (i, 0)),
        ],
        out_shape=[
            jax.ShapeDtypeStruct((NPAD, N_CLS), jnp.float32),
            jax.ShapeDtypeStruct((NPAD, N_CLS), jnp.float32),
            jax.ShapeDtypeStruct((NPAD, N_CLS), jnp.float32),
        ],
    )(aggp_r, aggp_k, h1, dinv_r, dinv_k, b1, W2)


def _tc3(aggp_r, aggp_k, h2, dinv_r, dinv_k, b2, Wl, bl):
    grid = NPAD // BLK

    def body(ar_ref, ak_ref, h2_ref, dvr_ref, dvk_ref, b2_ref, wl_ref, bl_ref,
             out_ref):
        dvr = dvr_ref[...]
        dvk = dvk_ref[...]
        h2 = h2_ref[...]
        b2 = b2_ref[...]
        conv_r = dvr * (ar_ref[0] + ar_ref[1]) + (dvr * dvr) * h2 + b2
        conv_k = dvk * (ak_ref[0] + ak_ref[1]) + (dvk * dvk) * h2 + b2
        r2 = jnp.concatenate([conv_r, conv_k], axis=1)
        final = jnp.dot(r2, wl_ref[...], preferred_element_type=jnp.float32)
        final = final + bl_ref[...]
        m = jnp.max(final, axis=1, keepdims=True)
        lse = jnp.log(jnp.sum(jnp.exp(final - m), axis=1, keepdims=True)) + m
        out_ref[...] = final - lse

    return pl.pallas_call(
        body,
        grid=(grid,),
        in_specs=[
            pl.BlockSpec((NC, BLK, N_CLS), lambda i: (0, i, 0)),
            pl.BlockSpec((NC, BLK, N_CLS), lambda i: (0, i, 0)),
            pl.BlockSpec((BLK, N_CLS), lambda i: (i, 0)),
            pl.BlockSpec((BLK, 1), lambda i: (i, 0)),
            pl.BlockSpec((BLK, 1), lambda i: (i, 0)),
            pl.BlockSpec((1, N_CLS), lambda i: (0, 0)),
            pl.BlockSpec((2 * N_CLS, N_CLS), lambda i: (0, 0)),
            pl.BlockSpec((1, N_CLS), lambda i: (0, 0)),
        ],
        out_specs=pl.BlockSpec((BLK, N_CLS), lambda i: (i, 0)),
        out_shape=jax.ShapeDtypeStruct((NPAD, N_CLS), jnp.float32),
    )(aggp_r, aggp_k, h2, dinv_r, dinv_k, b2, Wl, bl)


def kernel(x, edge_index, edge_index_knn, W1, b1, W2, b2, Wl, bl):
    e_r = edge_index.shape[1]
    e_k = edge_index_knn.shape[1]
    ch_r = _n_chunks(e_r)
    ch_k = _n_chunks(e_k)

    src_r = _pad_edges(edge_index[0], ch_r)
    dst_r = _pad_edges(edge_index[1], ch_r)
    src_k = _pad_edges(edge_index_knn[0], ch_k)
    dst_k = _pad_edges(edge_index_knn[1], ch_k)

    x_pad = jnp.zeros((NPAD, N_FEAT), jnp.float32).at[:N_NODES].set(x)
    ones16 = jnp.ones((CHUNK, 16), jnp.float32)
    zeros16 = jnp.zeros((NPAD, 16), jnp.float32)
    zeros_h = jnp.zeros((NPAD, N_HID), jnp.float32)
    zeros_c = jnp.zeros((NPAD, N_CLS), jnp.float32)
    b1r = b1.reshape(1, N_HID)
    b2r = b2.reshape(1, N_CLS)
    blr = bl.reshape(1, N_CLS)

    degp_r, degp_k = _make_deg_kernel(ch_r, ch_k)(dst_r, dst_k, ones16, zeros16)

    h1, hs1_r, hs1_k, dinv_r, dinv_k = _tc1(x_pad, W1, degp_r, degp_k)

    agg1 = _make_agg_kernel(N_HID, ch_r, ch_k)
    aggp1_r, aggp1_k = agg1(hs1_r, hs1_k, src_r, dst_r, src_k, dst_k, zeros_h)

    h2, hs2_r, hs2_k = _tc2(aggp1_r, aggp1_k, h1, dinv_r, dinv_k, b1r, W2)

    agg2 = _make_agg_kernel(N_CLS, ch_r, ch_k)
    aggp2_r, aggp2_k = agg2(hs2_r, hs2_k, src_r, dst_r, src_k, dst_k, zeros_c)

    out = _tc3(aggp2_r, aggp2_k, h2, dinv_r, dinv_k, b2r, Wl, blr)
    return out[:N_NODES]


# speedup vs baseline: 2.2984x; 1.1861x over previous
import functools

import jax
import jax.numpy as jnp
from jax import lax
from jax.experimental import pallas as pl
from jax.experimental.pallas import tpu as pltpu
from jax.experimental.pallas import tpu_sc as plsc

N_NODES = 10000
N_FEAT = 128
N_HID = 64
N_CLS = 32

NPAD = 10240
BLK = 1024
NW = 32
NC = 2
NS = 16
ROWS_PER_TILE = NPAD // NS
CHUNK = 128


def _n_chunks(e):
    ch = -(-e // (NW * CHUNK))
    return -(-ch // 4) * 4


def _pad_edges(idx, n_chunks):
    e = idx.shape[0]
    total = NW * n_chunks * CHUNK
    pad = N_NODES + (jnp.arange(total - e, dtype=jnp.int32) % (NPAD - N_NODES))
    return jnp.concatenate([idx.astype(jnp.int32), pad]).reshape(NW, n_chunks, CHUNK)


def _sc_mesh():
    return plsc.VectorSubcoreMesh(core_axis_name="c", subcore_axis_name="s",
                                  num_cores=NC, num_subcores=NS)


def _make_deg_kernel(ch_r, ch_k):
    out_t = (jax.ShapeDtypeStruct((NC, NPAD, 16), jnp.float32),
             jax.ShapeDtypeStruct((NC, NPAD, 16), jnp.float32))

    @functools.partial(
        pl.kernel,
        out_type=out_t,
        mesh=_sc_mesh(),
        compiler_params=pltpu.CompilerParams(use_tc_tiling_on_sc=False),
        scratch_types=[
            pltpu.VMEM((ch_r, CHUNK), jnp.int32),
            pltpu.VMEM((ch_k, CHUNK), jnp.int32),
            pltpu.VMEM((CHUNK, 16), jnp.float32),
            pltpu.VMEM_SHARED((NPAD, 16), jnp.float32),
            pltpu.VMEM_SHARED((NPAD, 16), jnp.float32),
        ],
    )
    def deg_kernel(dstr_hbm, dstk_hbm, ones_hbm, zeros_hbm, outr_hbm, outk_hbm,
                   dstr_v, dstk_v, ones_v, acc_r, acc_k):
        c = lax.axis_index("c")
        s = lax.axis_index("s")
        w = s * NC + c
        r0 = s * ROWS_PER_TILE
        pltpu.sync_copy(zeros_hbm.at[pl.ds(r0, ROWS_PER_TILE)],
                        acc_r.at[pl.ds(r0, ROWS_PER_TILE)])
        pltpu.sync_copy(zeros_hbm.at[pl.ds(r0, ROWS_PER_TILE)],
                        acc_k.at[pl.ds(r0, ROWS_PER_TILE)])
        pltpu.sync_copy(dstr_hbm.at[w], dstr_v)
        pltpu.sync_copy(dstk_hbm.at[w], dstk_v)
        pltpu.sync_copy(ones_hbm, ones_v)
        plsc.subcore_barrier()

        def body_r(j, carry):
            pltpu.sync_copy(ones_v, acc_r.at[dstr_v.at[j]], add=True)
            return carry

        lax.fori_loop(0, ch_r, body_r, 0)

        def body_k(j, carry):
            pltpu.sync_copy(ones_v, acc_k.at[dstk_v.at[j]], add=True)
            return carry

        lax.fori_loop(0, ch_k, body_k, 0)
        plsc.subcore_barrier()
        pltpu.sync_copy(acc_r.at[pl.ds(r0, ROWS_PER_TILE)],
                        outr_hbm.at[c].at[pl.ds(r0, ROWS_PER_TILE)])
        pltpu.sync_copy(acc_k.at[pl.ds(r0, ROWS_PER_TILE)],
                        outk_hbm.at[c].at[pl.ds(r0, ROWS_PER_TILE)])

    return deg_kernel


def _make_agg_kernel(feat, ch_r, ch_k):
    out_t = (jax.ShapeDtypeStruct((NC, NPAD, feat), jnp.float32),
             jax.ShapeDtypeStruct((NC, NPAD, feat), jnp.float32))

    @functools.partial(
        pl.kernel,
        out_type=out_t,
        mesh=_sc_mesh(),
        compiler_params=pltpu.CompilerParams(use_tc_tiling_on_sc=False),
        scratch_types=[
            pltpu.VMEM((ch_r, CHUNK), jnp.int32),
            pltpu.VMEM((ch_r, CHUNK), jnp.int32),
            pltpu.VMEM((ch_k, CHUNK), jnp.int32),
            pltpu.VMEM((ch_k, CHUNK), jnp.int32),
            [pltpu.VMEM((CHUNK, feat), jnp.float32) for _ in range(4)],
            pltpu.VMEM_SHARED((NPAD, feat), jnp.float32),
            [pltpu.SemaphoreType.DMA for _ in range(4)],
            [pltpu.SemaphoreType.DMA for _ in range(4)],
        ],
    )
    def agg_kernel(table_r, table_k, srcr_hbm, dstr_hbm, srck_hbm, dstk_hbm,
                   zeros_hbm, outr_hbm, outk_hbm,
                   srcr_v, dstr_v, srck_v, dstk_v, bufs, acc,
                   sem_g, sem_s):
        c = lax.axis_index("c")
        s = lax.axis_index("s")
        w = s * NC + c
        r0 = s * ROWS_PER_TILE
        pltpu.sync_copy(zeros_hbm.at[pl.ds(r0, ROWS_PER_TILE)],
                        acc.at[pl.ds(r0, ROWS_PER_TILE)])
        pltpu.sync_copy(srcr_hbm.at[w], srcr_v)
        pltpu.sync_copy(dstr_hbm.at[w], dstr_v)
        pltpu.sync_copy(srck_hbm.at[w], srck_v)
        pltpu.sync_copy(dstk_hbm.at[w], dstk_v)
        plsc.subcore_barrier()

        def pipelined(src_v, dst_v, table, n_chunks):
            pltpu.async_copy(table.at[src_v.at[0]], bufs[0], sem_g[0])
            pltpu.async_copy(table.at[src_v.at[1]], bufs[1], sem_g[1])

            def body(g, carry):
                for b in range(4):
                    j = 4 * g + b
                    c2 = (b + 2) % 4
                    pltpu.make_async_copy(table.at[src_v.at[j]], bufs[b],
                                          sem_g[b]).wait()
                    pltpu.async_copy(bufs[b], acc.at[dst_v.at[j]], sem_s[b],
                                     add=True)

                    @pl.when(j >= 2)
                    def _():
                        pltpu.make_async_copy(
                            bufs[c2], acc.at[dst_v.at[j - 2]], sem_s[c2]
                        ).wait()

                    @pl.when(j + 2 < n_chunks)
                    def _():
                        pltpu.async_copy(table.at[src_v.at[j + 2]], bufs[c2],
                                         sem_g[c2])
                return carry

            lax.fori_loop(0, n_chunks // 4, body, 0)
            pltpu.make_async_copy(bufs[2], acc.at[dst_v.at[n_chunks - 2]],
                                  sem_s[2]).wait()
            pltpu.make_async_copy(bufs[3], acc.at[dst_v.at[n_chunks - 1]],
                                  sem_s[3]).wait()

        def flush(out_hbm):
            plsc.subcore_barrier()
            pltpu.sync_copy(acc.at[pl.ds(r0, ROWS_PER_TILE)],
                            out_hbm.at[c].at[pl.ds(r0, ROWS_PER_TILE)])
            pltpu.sync_copy(zeros_hbm.at[pl.ds(r0, ROWS_PER_TILE)],
                            acc.at[pl.ds(r0, ROWS_PER_TILE)])
            plsc.subcore_barrier()

        pipelined(srcr_v, dstr_v, table_r, ch_r)
        flush(outr_hbm)
        pipelined(srck_v, dstk_v, table_k, ch_k)
        plsc.subcore_barrier()
        pltpu.sync_copy(acc.at[pl.ds(r0, ROWS_PER_TILE)],
                        outk_hbm.at[c].at[pl.ds(r0, ROWS_PER_TILE)])

    return agg_kernel


def _tc1(x_pad, W1, degp_r, degp_k):
    grid = NPAD // BLK

    def body(x_ref, w_ref, dr_ref, dk_ref,
             h1_ref, hsr_ref, hsk_ref, dvr_ref, dvk_ref):
        h1 = jnp.dot(x_ref[...], w_ref[...], preferred_element_type=jnp.float32)
        deg_r = dr_ref[0, :, 0:1] + dr_ref[1, :, 0:1] + 1.0
        deg_k = dk_ref[0, :, 0:1] + dk_ref[1, :, 0:1] + 1.0
        dinv_r = lax.rsqrt(deg_r)
        dinv_k = lax.rsqrt(deg_k)
        h1_ref[...] = h1
        hsr_ref[...] = h1 * dinv_r
        hsk_ref[...] = h1 * dinv_k
        dvr_ref[...] = dinv_r
        dvk_ref[...] = dinv_k

    return pl.pallas_call(
        body,
        grid=(grid,),
        in_specs=[
            pl.BlockSpec((BLK, N_FEAT), lambda i: (i, 0)),
            pl.BlockSpec((N_FEAT, N_HID), lambda i: (0, 0)),
            pl.BlockSpec((NC, BLK, 16), lambda i: (0, i, 0)),
            pl.BlockSpec((NC, BLK, 16), lambda i: (0, i, 0)),
        ],
        out_specs=[
            pl.BlockSpec((BLK, N_HID), lambda i: (i, 0)),
            pl.BlockSpec((BLK, N_HID), lambda i: (i, 0)),
            pl.BlockSpec((BLK, N_HID), lambda i: (i, 0)),
            pl.BlockSpec((BLK, 1), lambda i: (i, 0)),
            pl.BlockSpec((BLK, 1), lambda i: (i, 0)),
        ],
        out_shape=[
            jax.ShapeDtypeStruct((NPAD, N_HID), jnp.float32),
            jax.ShapeDtypeStruct((NPAD, N_HID), jnp.float32),
            jax.ShapeDtypeStruct((NPAD, N_HID), jnp.float32),
            jax.ShapeDtypeStruct((NPAD, 1), jnp.float32),
            jax.ShapeDtypeStruct((NPAD, 1), jnp.float32),
        ],
    )(x_pad, W1, degp_r, degp_k)


def _tc2(aggp_r, aggp_k, h1, dinv_r, dinv_k, b1, W2):
    grid = NPAD // BLK

    def body(ar_ref, ak_ref, h1_ref, dvr_ref, dvk_ref, b1_ref, w2_ref,
             h2_ref, hsr_ref, hsk_ref):
        dvr = dvr_ref[...]
        dvk = dvk_ref[...]
        h1 = h1_ref[...]
        b1 = b1_ref[...]
        conv_r = dvr * (ar_ref[0] + ar_ref[1]) + (dvr * dvr) * h1 + b1
        conv_k = dvk * (ak_ref[0] + ak_ref[1]) + (dvk * dvk) * h1 + b1
        r1 = jax.nn.relu(jnp.concatenate([conv_r, conv_k], axis=1))
        h2 = jnp.dot(r1, w2_ref[...], preferred_element_type=jnp.float32)
        h2_ref[...] = h2
        hsr_ref[...] = h2 * dvr
        hsk_ref[...] = h2 * dvk

    return pl.pallas_call(
        body,
        grid=(grid,),
        in_specs=[
            pl.BlockSpec((NC, BLK, N_HID), lambda i: (0, i, 0)),
            pl.BlockSpec((NC, BLK, N_HID), lambda i: (0, i, 0)),
            pl.BlockSpec((BLK, N_HID), lambda i: (i, 0)),
            pl.BlockSpec((BLK, 1), lambda i: (i, 0)),
            pl.BlockSpec((BLK, 1), lambda i: (i, 0)),
            pl.BlockSpec((1, N_HID), lambda i: (0, 0)),
            pl.BlockSpec((2 * N_HID, N_CLS), lambda i: (0, 0)),
        ],
        out_specs=[
            pl.BlockSpec((BLK, N_CLS), lambda i: (i, 0)),
            pl.BlockSpec((BLK, N_CLS), lambda i: (i, 0)),
            pl.BlockSpec((BLK, N_CLS), lambda i: (i, 0)),
        ],
        out_shape=[
            jax.ShapeDtypeStruct((NPAD, N_CLS), jnp.float32),
            jax.ShapeDtypeStruct((NPAD, N_CLS), jnp.float32),
            jax.ShapeDtypeStruct((NPAD, N_CLS), jnp.float32),
        ],
    )(aggp_r, aggp_k, h1, dinv_r, dinv_k, b1, W2)


def _tc3(aggp_r, aggp_k, h2, dinv_r, dinv_k, b2, Wl, bl):
    grid = NPAD // BLK

    def body(ar_ref, ak_ref, h2_ref, dvr_ref, dvk_ref, b2_ref, wl_ref, bl_ref,
             out_ref):
        dvr = dvr_ref[...]
        dvk = dvk_ref[...]
        h2 = h2_ref[...]
        b2 = b2_ref[...]
        conv_r = dvr * (ar_ref[0] + ar_ref[1]) + (dvr * dvr) * h2 + b2
        conv_k = dvk * (ak_ref[0] + ak_ref[1]) + (dvk * dvk) * h2 + b2
        r2 = jnp.concatenate([conv_r, conv_k], axis=1)
        final = jnp.dot(r2, wl_ref[...], preferred_element_type=jnp.float32)
        final = final + bl_ref[...]
        m = jnp.max(final, axis=1, keepdims=True)
        lse = jnp.log(jnp.sum(jnp.exp(final - m), axis=1, keepdims=True)) + m
        out_ref[...] = final - lse

    return pl.pallas_call(
        body,
        grid=(grid,),
        in_specs=[
            pl.BlockSpec((NC, BLK, N_CLS), lambda i: (0, i, 0)),
            pl.BlockSpec((NC, BLK, N_CLS), lambda i: (0, i, 0)),
            pl.BlockSpec((BLK, N_CLS), lambda i: (i, 0)),
            pl.BlockSpec((BLK, 1), lambda i: (i, 0)),
            pl.BlockSpec((BLK, 1), lambda i: (i, 0)),
            pl.BlockSpec((1, N_CLS), lambda i: (0, 0)),
            pl.BlockSpec((2 * N_CLS, N_CLS), lambda i: (0, 0)),
            pl.BlockSpec((1, N_CLS), lambda i: (0, 0)),
        ],
        out_specs=pl.BlockSpec((BLK, N_CLS), lambda i: (i, 0)),
        out_shape=jax.ShapeDtypeStruct((NPAD, N_CLS), jnp.float32),
    )(aggp_r, aggp_k, h2, dinv_r, dinv_k, b2, Wl, bl)


def kernel(x, edge_index, edge_index_knn, W1, b1, W2, b2, Wl, bl):
    e_r = edge_index.shape[1]
    e_k = edge_index_knn.shape[1]
    ch_r = _n_chunks(e_r)
    ch_k = _n_chunks(e_k)

    src_r = _pad_edges(edge_index[0], ch_r)
    dst_r = _pad_edges(edge_index[1], ch_r)
    src_k = _pad_edges(edge_index_knn[0], ch_k)
    dst_k = _pad_edges(edge_index_knn[1], ch_k)

    x_pad = jnp.zeros((NPAD, N_FEAT), jnp.float32).at[:N_NODES].set(x)
    ones16 = jnp.ones((CHUNK, 16), jnp.float32)
    zeros16 = jnp.zeros((NPAD, 16), jnp.float32)
    zeros_h = jnp.zeros((NPAD, N_HID), jnp.float32)
    zeros_c = jnp.zeros((NPAD, N_CLS), jnp.float32)
    b1r = b1.reshape(1, N_HID)
    b2r = b2.reshape(1, N_CLS)
    blr = bl.reshape(1, N_CLS)

    degp_r, degp_k = _make_deg_kernel(ch_r, ch_k)(dst_r, dst_k, ones16, zeros16)

    h1, hs1_r, hs1_k, dinv_r, dinv_k = _tc1(x_pad, W1, degp_r, degp_k)

    agg1 = _make_agg_kernel(N_HID, ch_r, ch_k)
    aggp1_r, aggp1_k = agg1(hs1_r, hs1_k, src_r, dst_r, src_k, dst_k, zeros_h)

    h2, hs2_r, hs2_k = _tc2(aggp1_r, aggp1_k, h1, dinv_r, dinv_k, b1r, W2)

    agg2 = _make_agg_kernel(N_CLS, ch_r, ch_k)
    aggp2_r, aggp2_k = agg2(hs2_r, hs2_k, src_r, dst_r, src_k, dst_k, zeros_c)

    out = _tc3(aggp2_r, aggp2_k, h2, dinv_r, dinv_k, b2r, Wl, blr)
    return out[:N_NODES]


# speedup vs baseline: 2.5451x; 1.1073x over previous
import functools

import jax
import jax.numpy as jnp
from jax import lax
from jax.experimental import pallas as pl
from jax.experimental.pallas import tpu as pltpu
from jax.experimental.pallas import tpu_sc as plsc

N_NODES = 10000
N_FEAT = 128
N_HID = 64
N_CLS = 32

NPAD = 10240
BLK = 1024
NW = 32
NC = 2
NS = 16
ROWS_PER_TILE = NPAD // NS
CHUNK = 128


def _n_chunks(e):
    ch = -(-e // (NW * CHUNK))
    return -(-ch // 4) * 4


def _pad_edges(idx, n_chunks):
    e = idx.shape[0]
    total = NW * n_chunks * CHUNK
    pad = N_NODES + (jnp.arange(total - e, dtype=jnp.int32) % (NPAD - N_NODES))
    return jnp.concatenate([idx.astype(jnp.int32), pad]).reshape(NW, n_chunks, CHUNK)


def _sc_mesh():
    return plsc.VectorSubcoreMesh(core_axis_name="c", subcore_axis_name="s",
                                  num_cores=NC, num_subcores=NS)


def _make_deg_kernel(ch_r, ch_k):
    out_t = (jax.ShapeDtypeStruct((NC, NPAD, 128), jnp.float32),
             jax.ShapeDtypeStruct((NC, NPAD, 128), jnp.float32))

    @functools.partial(
        pl.kernel,
        out_type=out_t,
        mesh=_sc_mesh(),
        compiler_params=pltpu.CompilerParams(use_tc_tiling_on_sc=False),
        scratch_types=[
            pltpu.VMEM((ch_r, CHUNK), jnp.int32),
            pltpu.VMEM((ch_k, CHUNK), jnp.int32),
            pltpu.VMEM((CHUNK, 16), jnp.float32),
            pltpu.VMEM_SHARED((NPAD, 16), jnp.float32),
            pltpu.VMEM_SHARED((NPAD, 16), jnp.float32),
        ],
    )
    def deg_kernel(dstr_hbm, dstk_hbm, ones_hbm, zeros_hbm, outr_hbm, outk_hbm,
                   dstr_v, dstk_v, ones_v, acc_r, acc_k):
        c = lax.axis_index("c")
        s = lax.axis_index("s")
        w = s * NC + c
        r0 = s * ROWS_PER_TILE
        pltpu.sync_copy(zeros_hbm.at[pl.ds(r0, ROWS_PER_TILE)],
                        acc_r.at[pl.ds(r0, ROWS_PER_TILE)])
        pltpu.sync_copy(zeros_hbm.at[pl.ds(r0, ROWS_PER_TILE)],
                        acc_k.at[pl.ds(r0, ROWS_PER_TILE)])
        pltpu.sync_copy(dstr_hbm.at[w], dstr_v)
        pltpu.sync_copy(dstk_hbm.at[w], dstk_v)
        pltpu.sync_copy(ones_hbm, ones_v)
        plsc.subcore_barrier()

        def body_r(j, carry):
            pltpu.sync_copy(ones_v, acc_r.at[dstr_v.at[j]], add=True)
            return carry

        lax.fori_loop(0, ch_r, body_r, 0)

        def body_k(j, carry):
            pltpu.sync_copy(ones_v, acc_k.at[dstk_v.at[j]], add=True)
            return carry

        lax.fori_loop(0, ch_k, body_k, 0)
        plsc.subcore_barrier()
        pltpu.sync_copy(acc_r.at[pl.ds(r0, ROWS_PER_TILE)],
                        outr_hbm.at[c].at[pl.ds(r0, ROWS_PER_TILE), pl.ds(0, 16)])
        pltpu.sync_copy(acc_k.at[pl.ds(r0, ROWS_PER_TILE)],
                        outk_hbm.at[c].at[pl.ds(r0, ROWS_PER_TILE), pl.ds(0, 16)])

    return deg_kernel


def _make_agg_kernel(feat, ch_r, ch_k):
    out_t = (jax.ShapeDtypeStruct((NC, NPAD, 128), jnp.float32),
             jax.ShapeDtypeStruct((NC, NPAD, 128), jnp.float32))

    @functools.partial(
        pl.kernel,
        out_type=out_t,
        mesh=_sc_mesh(),
        compiler_params=pltpu.CompilerParams(use_tc_tiling_on_sc=False),
        scratch_types=[
            pltpu.VMEM((ch_r, CHUNK), jnp.int32),
            pltpu.VMEM((ch_r, CHUNK), jnp.int32),
            pltpu.VMEM((ch_k, CHUNK), jnp.int32),
            pltpu.VMEM((ch_k, CHUNK), jnp.int32),
            [pltpu.VMEM((CHUNK, feat), jnp.float32) for _ in range(4)],
            pltpu.VMEM_SHARED((NPAD, feat), jnp.float32),
            [pltpu.SemaphoreType.DMA for _ in range(4)],
            [pltpu.SemaphoreType.DMA for _ in range(4)],
        ],
    )
    def agg_kernel(table_r, table_k, srcr_hbm, dstr_hbm, srck_hbm, dstk_hbm,
                   zeros_hbm, outr_hbm, outk_hbm,
                   srcr_v, dstr_v, srck_v, dstk_v, bufs, acc,
                   sem_g, sem_s):
        c = lax.axis_index("c")
        s = lax.axis_index("s")
        w = s * NC + c
        r0 = s * ROWS_PER_TILE
        pltpu.sync_copy(zeros_hbm.at[pl.ds(r0, ROWS_PER_TILE)],
                        acc.at[pl.ds(r0, ROWS_PER_TILE)])
        pltpu.sync_copy(srcr_hbm.at[w], srcr_v)
        pltpu.sync_copy(dstr_hbm.at[w], dstr_v)
        pltpu.sync_copy(srck_hbm.at[w], srck_v)
        pltpu.sync_copy(dstk_hbm.at[w], dstk_v)
        plsc.subcore_barrier()

        def pipelined(src_v, dst_v, table, n_chunks):
            pltpu.async_copy(table.at[src_v.at[0]], bufs[0], sem_g[0])
            pltpu.async_copy(table.at[src_v.at[1]], bufs[1], sem_g[1])

            def body(g, carry):
                for b in range(4):
                    j = 4 * g + b
                    c2 = (b + 2) % 4
                    pltpu.make_async_copy(table.at[src_v.at[j]], bufs[b],
                                          sem_g[b]).wait()
                    pltpu.async_copy(bufs[b], acc.at[dst_v.at[j]], sem_s[b],
                                     add=True)

                    @pl.when(j >= 2)
                    def _():
                        pltpu.make_async_copy(
                            bufs[c2], acc.at[dst_v.at[j - 2]], sem_s[c2]
                        ).wait()

                    @pl.when(j + 2 < n_chunks)
                    def _():
                        pltpu.async_copy(table.at[src_v.at[j + 2]], bufs[c2],
                                         sem_g[c2])
                return carry

            lax.fori_loop(0, n_chunks // 4, body, 0)
            pltpu.make_async_copy(bufs[2], acc.at[dst_v.at[n_chunks - 2]],
                                  sem_s[2]).wait()
            pltpu.make_async_copy(bufs[3], acc.at[dst_v.at[n_chunks - 1]],
                                  sem_s[3]).wait()

        def flush(out_hbm):
            plsc.subcore_barrier()
            pltpu.sync_copy(acc.at[pl.ds(r0, ROWS_PER_TILE)],
                            out_hbm.at[c].at[pl.ds(r0, ROWS_PER_TILE),
                                             pl.ds(0, feat)])
            pltpu.sync_copy(zeros_hbm.at[pl.ds(r0, ROWS_PER_TILE)],
                            acc.at[pl.ds(r0, ROWS_PER_TILE)])
            plsc.subcore_barrier()

        pipelined(srcr_v, dstr_v, table_r, ch_r)
        flush(outr_hbm)
        pipelined(srck_v, dstk_v, table_k, ch_k)
        plsc.subcore_barrier()
        pltpu.sync_copy(acc.at[pl.ds(r0, ROWS_PER_TILE)],
                        outk_hbm.at[c].at[pl.ds(r0, ROWS_PER_TILE),
                                          pl.ds(0, feat)])

    return agg_kernel


def _tc1(x_pad, W1, degp_r, degp_k):
    grid = NPAD // BLK

    def body(x_ref, w_ref, dr_ref, dk_ref,
             h1_ref, hsr_ref, hsk_ref, dvr_ref, dvk_ref):
        h1 = jnp.dot(x_ref[...], w_ref[...], preferred_element_type=jnp.float32)
        deg_r = dr_ref[0, :, 0:1] + dr_ref[1, :, 0:1] + 1.0
        deg_k = dk_ref[0, :, 0:1] + dk_ref[1, :, 0:1] + 1.0
        dinv_r = lax.rsqrt(deg_r)
        dinv_k = lax.rsqrt(deg_k)
        h1_ref[...] = h1
        hsr_ref[...] = h1 * dinv_r
        hsk_ref[...] = h1 * dinv_k
        dvr_ref[...] = dinv_r
        dvk_ref[...] = dinv_k

    return pl.pallas_call(
        body,
        grid=(grid,),
        in_specs=[
            pl.BlockSpec((BLK, N_FEAT), lambda i: (i, 0)),
            pl.BlockSpec((N_FEAT, N_HID), lambda i: (0, 0)),
            pl.BlockSpec((NC, BLK, 128), lambda i: (0, i, 0)),
            pl.BlockSpec((NC, BLK, 128), lambda i: (0, i, 0)),
        ],
        out_specs=[
            pl.BlockSpec((BLK, N_HID), lambda i: (i, 0)),
            pl.BlockSpec((BLK, N_HID), lambda i: (i, 0)),
            pl.BlockSpec((BLK, N_HID), lambda i: (i, 0)),
            pl.BlockSpec((BLK, 1), lambda i: (i, 0)),
            pl.BlockSpec((BLK, 1), lambda i: (i, 0)),
        ],
        out_shape=[
            jax.ShapeDtypeStruct((NPAD, N_HID), jnp.float32),
            jax.ShapeDtypeStruct((NPAD, N_HID), jnp.float32),
            jax.ShapeDtypeStruct((NPAD, N_HID), jnp.float32),
            jax.ShapeDtypeStruct((NPAD, 1), jnp.float32),
            jax.ShapeDtypeStruct((NPAD, 1), jnp.float32),
        ],
    )(x_pad, W1, degp_r, degp_k)


def _tc2(aggp_r, aggp_k, h1, dinv_r, dinv_k, b1, W2):
    grid = NPAD // BLK

    def body(ar_ref, ak_ref, h1_ref, dvr_ref, dvk_ref, b1_ref, w2_ref,
             h2_ref, hsr_ref, hsk_ref):
        dvr = dvr_ref[...]
        dvk = dvk_ref[...]
        h1 = h1_ref[...]
        b1 = b1_ref[...]
        agg_r = ar_ref[0, :, :N_HID] + ar_ref[1, :, :N_HID]
        agg_k = ak_ref[0, :, :N_HID] + ak_ref[1, :, :N_HID]
        conv_r = dvr * agg_r + (dvr * dvr) * h1 + b1
        conv_k = dvk * agg_k + (dvk * dvk) * h1 + b1
        r1 = jax.nn.relu(jnp.concatenate([conv_r, conv_k], axis=1))
        h2 = jnp.dot(r1, w2_ref[...], preferred_element_type=jnp.float32)
        h2_ref[...] = h2
        hsr_ref[...] = h2 * dvr
        hsk_ref[...] = h2 * dvk

    return pl.pallas_call(
        body,
        grid=(grid,),
        in_specs=[
            pl.BlockSpec((NC, BLK, 128), lambda i: (0, i, 0)),
            pl.BlockSpec((NC, BLK, 128), lambda i: (0, i, 0)),
            pl.BlockSpec((BLK, N_HID), lambda i: (i, 0)),
            pl.BlockSpec((BLK, 1), lambda i: (i, 0)),
            pl.BlockSpec((BLK, 1), lambda i: (i, 0)),
            pl.BlockSpec((1, N_HID), lambda i: (0, 0)),
            pl.BlockSpec((2 * N_HID, N_CLS), lambda i: (0, 0)),
        ],
        out_specs=[
            pl.BlockSpec((BLK, N_CLS), lambda i: (i, 0)),
            pl.BlockSpec((BLK, N_CLS), lambda i: (i, 0)),
            pl.BlockSpec((BLK, N_CLS), lambda i: (i, 0)),
        ],
        out_shape=[
            jax.ShapeDtypeStruct((NPAD, N_CLS), jnp.float32),
            jax.ShapeDtypeStruct((NPAD, N_CLS), jnp.float32),
            jax.ShapeDtypeStruct((NPAD, N_CLS), jnp.float32),
        ],
    )(aggp_r, aggp_k, h1, dinv_r, dinv_k, b1, W2)


def _tc3(aggp_r, aggp_k, h2, dinv_r, dinv_k, b2, Wl, bl):
    grid = NPAD // BLK

    def body(ar_ref, ak_ref, h2_ref, dvr_ref, dvk_ref, b2_ref, wl_ref, bl_ref,
             out_ref):
        dvr = dvr_ref[...]
        dvk = dvk_ref[...]
        h2 = h2_ref[...]
        b2 = b2_ref[...]
        agg_r = ar_ref[0, :, :N_CLS] + ar_ref[1, :, :N_CLS]
        agg_k = ak_ref[0, :, :N_CLS] + ak_ref[1, :, :N_CLS]
        conv_r = dvr * agg_r + (dvr * dvr) * h2 + b2
        conv_k = dvk * agg_k + (dvk * dvk) * h2 + b2
        r2 = jnp.concatenate([conv_r, conv_k], axis=1)
        final = jnp.dot(r2, wl_ref[...], preferred_element_type=jnp.float32)
        final = final + bl_ref[...]
        m = jnp.max(final, axis=1, keepdims=True)
        lse = jnp.log(jnp.sum(jnp.exp(final - m), axis=1, keepdims=True)) + m
        out_ref[...] = final - lse

    return pl.pallas_call(
        body,
        grid=(grid,),
        in_specs=[
            pl.BlockSpec((NC, BLK, 128), lambda i: (0, i, 0)),
            pl.BlockSpec((NC, BLK, 128), lambda i: (0, i, 0)),
            pl.BlockSpec((BLK, N_CLS), lambda i: (i, 0)),
            pl.BlockSpec((BLK, 1), lambda i: (i, 0)),
            pl.BlockSpec((BLK, 1), lambda i: (i, 0)),
            pl.BlockSpec((1, N_CLS), lambda i: (0, 0)),
            pl.BlockSpec((2 * N_CLS, N_CLS), lambda i: (0, 0)),
            pl.BlockSpec((1, N_CLS), lambda i: (0, 0)),
        ],
        out_specs=pl.BlockSpec((BLK, N_CLS), lambda i: (i, 0)),
        out_shape=jax.ShapeDtypeStruct((NPAD, N_CLS), jnp.float32),
    )(aggp_r, aggp_k, h2, dinv_r, dinv_k, b2, Wl, bl)


def kernel(x, edge_index, edge_index_knn, W1, b1, W2, b2, Wl, bl):
    e_r = edge_index.shape[1]
    e_k = edge_index_knn.shape[1]
    ch_r = _n_chunks(e_r)
    ch_k = _n_chunks(e_k)

    src_r = _pad_edges(edge_index[0], ch_r)
    dst_r = _pad_edges(edge_index[1], ch_r)
    src_k = _pad_edges(edge_index_knn[0], ch_k)
    dst_k = _pad_edges(edge_index_knn[1], ch_k)

    x_pad = jnp.zeros((NPAD, N_FEAT), jnp.float32).at[:N_NODES].set(x)
    ones16 = jnp.ones((CHUNK, 16), jnp.float32)
    zeros16 = jnp.zeros((NPAD, 16), jnp.float32)
    zeros_h = jnp.zeros((NPAD, N_HID), jnp.float32)
    zeros_c = jnp.zeros((NPAD, N_CLS), jnp.float32)
    b1r = b1.reshape(1, N_HID)
    b2r = b2.reshape(1, N_CLS)
    blr = bl.reshape(1, N_CLS)

    degp_r, degp_k = _make_deg_kernel(ch_r, ch_k)(dst_r, dst_k, ones16, zeros16)

    h1, hs1_r, hs1_k, dinv_r, dinv_k = _tc1(x_pad, W1, degp_r, degp_k)

    agg1 = _make_agg_kernel(N_HID, ch_r, ch_k)
    aggp1_r, aggp1_k = agg1(hs1_r, hs1_k, src_r, dst_r, src_k, dst_k, zeros_h)

    h2, hs2_r, hs2_k = _tc2(aggp1_r, aggp1_k, h1, dinv_r, dinv_k, b1r, W2)

    agg2 = _make_agg_kernel(N_CLS, ch_r, ch_k)
    aggp2_r, aggp2_k = agg2(hs2_r, hs2_k, src_r, dst_r, src_k, dst_k, zeros_c)

    out = _tc3(aggp2_r, aggp2_k, h2, dinv_r, dinv_k, b2r, Wl, blr)
    return out[:N_NODES]


# speedup vs baseline: 2.5636x; 1.0073x over previous
import functools

import jax
import jax.numpy as jnp
from jax import lax
from jax.experimental import pallas as pl
from jax.experimental.pallas import tpu as pltpu
from jax.experimental.pallas import tpu_sc as plsc

N_NODES = 10000
N_FEAT = 128
N_HID = 64
N_CLS = 32

NPAD = 10240
BLK = 2048
NW = 32
NC = 2
NS = 16
ROWS_PER_TILE = NPAD // NS
CHUNK = 128


def _n_chunks(e):
    ch = -(-e // (NW * CHUNK))
    return -(-ch // 4) * 4


def _pad_edges(idx, n_chunks):
    e = idx.shape[0]
    total = NW * n_chunks * CHUNK
    pad = N_NODES + (jnp.arange(total - e, dtype=jnp.int32) % (NPAD - N_NODES))
    return jnp.concatenate([idx.astype(jnp.int32), pad]).reshape(NW, n_chunks, CHUNK)


def _sc_mesh():
    return plsc.VectorSubcoreMesh(core_axis_name="c", subcore_axis_name="s",
                                  num_cores=NC, num_subcores=NS)


def _make_deg_kernel(ch_r, ch_k):
    out_t = (jax.ShapeDtypeStruct((NC, NPAD, 128), jnp.float32),
             jax.ShapeDtypeStruct((NC, NPAD, 128), jnp.float32))

    @functools.partial(
        pl.kernel,
        out_type=out_t,
        mesh=_sc_mesh(),
        compiler_params=pltpu.CompilerParams(use_tc_tiling_on_sc=False),
        scratch_types=[
            pltpu.VMEM((ch_r, CHUNK), jnp.int32),
            pltpu.VMEM((ch_k, CHUNK), jnp.int32),
            pltpu.VMEM((CHUNK, 16), jnp.float32),
            pltpu.VMEM_SHARED((NPAD, 16), jnp.float32),
            pltpu.VMEM_SHARED((NPAD, 16), jnp.float32),
        ],
    )
    def deg_kernel(dstr_hbm, dstk_hbm, ones_hbm, zeros_hbm, outr_hbm, outk_hbm,
                   dstr_v, dstk_v, ones_v, acc_r, acc_k):
        c = lax.axis_index("c")
        s = lax.axis_index("s")
        w = s * NC + c
        r0 = s * ROWS_PER_TILE
        pltpu.sync_copy(zeros_hbm.at[pl.ds(r0, ROWS_PER_TILE)],
                        acc_r.at[pl.ds(r0, ROWS_PER_TILE)])
        pltpu.sync_copy(zeros_hbm.at[pl.ds(r0, ROWS_PER_TILE)],
                        acc_k.at[pl.ds(r0, ROWS_PER_TILE)])
        pltpu.sync_copy(dstr_hbm.at[w], dstr_v)
        pltpu.sync_copy(dstk_hbm.at[w], dstk_v)
        pltpu.sync_copy(ones_hbm, ones_v)
        plsc.subcore_barrier()

        def body_r(j, carry):
            pltpu.sync_copy(ones_v, acc_r.at[dstr_v.at[j]], add=True)
            return carry

        lax.fori_loop(0, ch_r, body_r, 0)

        def body_k(j, carry):
            pltpu.sync_copy(ones_v, acc_k.at[dstk_v.at[j]], add=True)
            return carry

        lax.fori_loop(0, ch_k, body_k, 0)
        plsc.subcore_barrier()
        pltpu.sync_copy(acc_r.at[pl.ds(r0, ROWS_PER_TILE)],
                        outr_hbm.at[c].at[pl.ds(r0, ROWS_PER_TILE), pl.ds(0, 16)])
        pltpu.sync_copy(acc_k.at[pl.ds(r0, ROWS_PER_TILE)],
                        outk_hbm.at[c].at[pl.ds(r0, ROWS_PER_TILE), pl.ds(0, 16)])

    return deg_kernel


def _make_agg_kernel(feat, ch_r, ch_k):
    out_t = (jax.ShapeDtypeStruct((NC, NPAD, 128), jnp.float32),
             jax.ShapeDtypeStruct((NC, NPAD, 128), jnp.float32))

    @functools.partial(
        pl.kernel,
        out_type=out_t,
        mesh=_sc_mesh(),
        compiler_params=pltpu.CompilerParams(use_tc_tiling_on_sc=False),
        scratch_types=[
            pltpu.VMEM((ch_r, CHUNK), jnp.int32),
            pltpu.VMEM((ch_r, CHUNK), jnp.int32),
            pltpu.VMEM((ch_k, CHUNK), jnp.int32),
            pltpu.VMEM((ch_k, CHUNK), jnp.int32),
            [pltpu.VMEM((CHUNK, feat), jnp.float32) for _ in range(4)],
            pltpu.VMEM_SHARED((NPAD, feat), jnp.float32),
            [pltpu.SemaphoreType.DMA for _ in range(4)],
            [pltpu.SemaphoreType.DMA for _ in range(4)],
        ],
    )
    def agg_kernel(table_r, table_k, srcr_hbm, dstr_hbm, srck_hbm, dstk_hbm,
                   zeros_hbm, outr_hbm, outk_hbm,
                   srcr_v, dstr_v, srck_v, dstk_v, bufs, acc,
                   sem_g, sem_s):
        c = lax.axis_index("c")
        s = lax.axis_index("s")
        w = s * NC + c
        r0 = s * ROWS_PER_TILE
        pltpu.sync_copy(zeros_hbm.at[pl.ds(r0, ROWS_PER_TILE)],
                        acc.at[pl.ds(r0, ROWS_PER_TILE)])
        pltpu.sync_copy(srcr_hbm.at[w], srcr_v)
        pltpu.sync_copy(dstr_hbm.at[w], dstr_v)
        pltpu.sync_copy(srck_hbm.at[w], srck_v)
        pltpu.sync_copy(dstk_hbm.at[w], dstk_v)
        plsc.subcore_barrier()

        def pipelined(src_v, dst_v, table, n_chunks):
            pltpu.async_copy(table.at[src_v.at[0]], bufs[0], sem_g[0])
            pltpu.async_copy(table.at[src_v.at[1]], bufs[1], sem_g[1])

            def body(g, carry):
                for b in range(4):
                    j = 4 * g + b
                    c2 = (b + 2) % 4
                    pltpu.make_async_copy(table.at[src_v.at[j]], bufs[b],
                                          sem_g[b]).wait()
                    pltpu.async_copy(bufs[b], acc.at[dst_v.at[j]], sem_s[b],
                                     add=True)

                    @pl.when(j >= 2)
                    def _():
                        pltpu.make_async_copy(
                            bufs[c2], acc.at[dst_v.at[j - 2]], sem_s[c2]
                        ).wait()

                    @pl.when(j + 2 < n_chunks)
                    def _():
                        pltpu.async_copy(table.at[src_v.at[j + 2]], bufs[c2],
                                         sem_g[c2])
                return carry

            lax.fori_loop(0, n_chunks // 4, body, 0)
            pltpu.make_async_copy(bufs[2], acc.at[dst_v.at[n_chunks - 2]],
                                  sem_s[2]).wait()
            pltpu.make_async_copy(bufs[3], acc.at[dst_v.at[n_chunks - 1]],
                                  sem_s[3]).wait()

        def flush(out_hbm):
            plsc.subcore_barrier()
            pltpu.sync_copy(acc.at[pl.ds(r0, ROWS_PER_TILE)],
                            out_hbm.at[c].at[pl.ds(r0, ROWS_PER_TILE),
                                             pl.ds(0, feat)])
            pltpu.sync_copy(zeros_hbm.at[pl.ds(r0, ROWS_PER_TILE)],
                            acc.at[pl.ds(r0, ROWS_PER_TILE)])
            plsc.subcore_barrier()

        pipelined(srcr_v, dstr_v, table_r, ch_r)
        flush(outr_hbm)
        pipelined(srck_v, dstk_v, table_k, ch_k)
        plsc.subcore_barrier()
        pltpu.sync_copy(acc.at[pl.ds(r0, ROWS_PER_TILE)],
                        outk_hbm.at[c].at[pl.ds(r0, ROWS_PER_TILE),
                                          pl.ds(0, feat)])

    return agg_kernel


def _tc1(x_pad, W1, degp_r, degp_k):
    grid = NPAD // BLK

    def body(x_ref, w_ref, dr_ref, dk_ref,
             h1_ref, hsr_ref, hsk_ref, dvr_ref, dvk_ref):
        h1 = jnp.dot(x_ref[...], w_ref[...], preferred_element_type=jnp.float32)
        deg_r = dr_ref[0, :, 0:1] + dr_ref[1, :, 0:1] + 1.0
        deg_k = dk_ref[0, :, 0:1] + dk_ref[1, :, 0:1] + 1.0
        dinv_r = lax.rsqrt(deg_r)
        dinv_k = lax.rsqrt(deg_k)
        h1_ref[...] = h1
        hsr_ref[...] = h1 * dinv_r
        hsk_ref[...] = h1 * dinv_k
        dvr_ref[...] = dinv_r
        dvk_ref[...] = dinv_k

    return pl.pallas_call(
        body,
        grid=(grid,),
        in_specs=[
            pl.BlockSpec((BLK, N_FEAT), lambda i: (i, 0)),
            pl.BlockSpec((N_FEAT, N_HID), lambda i: (0, 0)),
            pl.BlockSpec((NC, BLK, 128), lambda i: (0, i, 0)),
            pl.BlockSpec((NC, BLK, 128), lambda i: (0, i, 0)),
        ],
        out_specs=[
            pl.BlockSpec((BLK, N_HID), lambda i: (i, 0)),
            pl.BlockSpec((BLK, N_HID), lambda i: (i, 0)),
            pl.BlockSpec((BLK, N_HID), lambda i: (i, 0)),
            pl.BlockSpec((BLK, 1), lambda i: (i, 0)),
            pl.BlockSpec((BLK, 1), lambda i: (i, 0)),
        ],
        out_shape=[
            jax.ShapeDtypeStruct((NPAD, N_HID), jnp.float32),
            jax.ShapeDtypeStruct((NPAD, N_HID), jnp.float32),
            jax.ShapeDtypeStruct((NPAD, N_HID), jnp.float32),
            jax.ShapeDtypeStruct((NPAD, 1), jnp.float32),
            jax.ShapeDtypeStruct((NPAD, 1), jnp.float32),
        ],
    )(x_pad, W1, degp_r, degp_k)


def _tc2(aggp_r, aggp_k, h1, dinv_r, dinv_k, b1, W2):
    grid = NPAD // BLK

    def body(ar_ref, ak_ref, h1_ref, dvr_ref, dvk_ref, b1_ref, w2_ref,
             h2_ref, hsr_ref, hsk_ref):
        dvr = dvr_ref[...]
        dvk = dvk_ref[...]
        h1 = h1_ref[...]
        b1 = b1_ref[...]
        agg_r = ar_ref[0, :, :N_HID] + ar_ref[1, :, :N_HID]
        agg_k = ak_ref[0, :, :N_HID] + ak_ref[1, :, :N_HID]
        conv_r = dvr * agg_r + (dvr * dvr) * h1 + b1
        conv_k = dvk * agg_k + (dvk * dvk) * h1 + b1
        r1 = jax.nn.relu(jnp.concatenate([conv_r, conv_k], axis=1))
        h2 = jnp.dot(r1, w2_ref[...], preferred_element_type=jnp.float32)
        h2_ref[...] = h2
        hsr_ref[...] = h2 * dvr
        hsk_ref[...] = h2 * dvk

    return pl.pallas_call(
        body,
        grid=(grid,),
        in_specs=[
            pl.BlockSpec((NC, BLK, 128), lambda i: (0, i, 0)),
            pl.BlockSpec((NC, BLK, 128), lambda i: (0, i, 0)),
            pl.BlockSpec((BLK, N_HID), lambda i: (i, 0)),
            pl.BlockSpec((BLK, 1), lambda i: (i, 0)),
            pl.BlockSpec((BLK, 1), lambda i: (i, 0)),
            pl.BlockSpec((1, N_HID), lambda i: (0, 0)),
            pl.BlockSpec((2 * N_HID, N_CLS), lambda i: (0, 0)),
        ],
        out_specs=[
            pl.BlockSpec((BLK, N_CLS), lambda i: (i, 0)),
            pl.BlockSpec((BLK, N_CLS), lambda i: (i, 0)),
            pl.BlockSpec((BLK, N_CLS), lambda i: (i, 0)),
        ],
        out_shape=[
            jax.ShapeDtypeStruct((NPAD, N_CLS), jnp.float32),
            jax.ShapeDtypeStruct((NPAD, N_CLS), jnp.float32),
            jax.ShapeDtypeStruct((NPAD, N_CLS), jnp.float32),
        ],
    )(aggp_r, aggp_k, h1, dinv_r, dinv_k, b1, W2)


def _tc3(aggp_r, aggp_k, h2, dinv_r, dinv_k, b2, Wl, bl):
    grid = NPAD // BLK

    def body(ar_ref, ak_ref, h2_ref, dvr_ref, dvk_ref, b2_ref, wl_ref, bl_ref,
             out_ref):
        dvr = dvr_ref[...]
        dvk = dvk_ref[...]
        h2 = h2_ref[...]
        b2 = b2_ref[...]
        agg_r = ar_ref[0, :, :N_CLS] + ar_ref[1, :, :N_CLS]
        agg_k = ak_ref[0, :, :N_CLS] + ak_ref[1, :, :N_CLS]
        conv_r = dvr * agg_r + (dvr * dvr) * h2 + b2
        conv_k = dvk * agg_k + (dvk * dvk) * h2 + b2
        r2 = jnp.concatenate([conv_r, conv_k], axis=1)
        final = jnp.dot(r2, wl_ref[...], preferred_element_type=jnp.float32)
        final = final + bl_ref[...]
        m = jnp.max(final, axis=1, keepdims=True)
        lse = jnp.log(jnp.sum(jnp.exp(final - m), axis=1, keepdims=True)) + m
        out_ref[...] = final - lse

    return pl.pallas_call(
        body,
        grid=(grid,),
        in_specs=[
            pl.BlockSpec((NC, BLK, 128), lambda i: (0, i, 0)),
            pl.BlockSpec((NC, BLK, 128), lambda i: (0, i, 0)),
            pl.BlockSpec((BLK, N_CLS), lambda i: (i, 0)),
            pl.BlockSpec((BLK, 1), lambda i: (i, 0)),
            pl.BlockSpec((BLK, 1), lambda i: (i, 0)),
            pl.BlockSpec((1, N_CLS), lambda i: (0, 0)),
            pl.BlockSpec((2 * N_CLS, N_CLS), lambda i: (0, 0)),
            pl.BlockSpec((1, N_CLS), lambda i: (0, 0)),
        ],
        out_specs=pl.BlockSpec((BLK, N_CLS), lambda i: (i, 0)),
        out_shape=jax.ShapeDtypeStruct((NPAD, N_CLS), jnp.float32),
    )(aggp_r, aggp_k, h2, dinv_r, dinv_k, b2, Wl, bl)


def kernel(x, edge_index, edge_index_knn, W1, b1, W2, b2, Wl, bl):
    e_r = edge_index.shape[1]
    e_k = edge_index_knn.shape[1]
    ch_r = _n_chunks(e_r)
    ch_k = _n_chunks(e_k)

    src_r = _pad_edges(edge_index[0], ch_r)
    dst_r = _pad_edges(edge_index[1], ch_r)
    src_k = _pad_edges(edge_index_knn[0], ch_k)
    dst_k = _pad_edges(edge_index_knn[1], ch_k)

    x_pad = jnp.zeros((NPAD, N_FEAT), jnp.float32).at[:N_NODES].set(x)
    ones16 = jnp.ones((CHUNK, 16), jnp.float32)
    zeros16 = jnp.zeros((NPAD, 16), jnp.float32)
    zeros_h = jnp.zeros((NPAD, N_HID), jnp.float32)
    zeros_c = jnp.zeros((NPAD, N_CLS), jnp.float32)
    b1r = b1.reshape(1, N_HID)
    b2r = b2.reshape(1, N_CLS)
    blr = bl.reshape(1, N_CLS)

    degp_r, degp_k = _make_deg_kernel(ch_r, ch_k)(dst_r, dst_k, ones16, zeros16)

    h1, hs1_r, hs1_k, dinv_r, dinv_k = _tc1(x_pad, W1, degp_r, degp_k)

    agg1 = _make_agg_kernel(N_HID, ch_r, ch_k)
    aggp1_r, aggp1_k = agg1(hs1_r, hs1_k, src_r, dst_r, src_k, dst_k, zeros_h)

    h2, hs2_r, hs2_k = _tc2(aggp1_r, aggp1_k, h1, dinv_r, dinv_k, b1r, W2)

    agg2 = _make_agg_kernel(N_CLS, ch_r, ch_k)
    aggp2_r, aggp2_k = agg2(hs2_r, hs2_k, src_r, dst_r, src_k, dst_k, zeros_c)

    out = _tc3(aggp2_r, aggp2_k, h2, dinv_r, dinv_k, b2r, Wl, blr)
    return out[:N_NODES]


# speedup vs baseline: 2.5708x; 1.0028x over previous
import functools

import jax
import jax.numpy as jnp
from jax import lax
from jax.experimental import pallas as pl
from jax.experimental.pallas import tpu as pltpu
from jax.experimental.pallas import tpu_sc as plsc

N_NODES = 10000
N_FEAT = 128
N_HID = 64
N_CLS = 32

NPAD = 10240
BLK = 2048
NW = 32
NC = 2
NS = 16
ROWS_PER_TILE = NPAD // NS
CHUNK = 128


def _n_chunks(e):
    ch = -(-e // (NW * CHUNK))
    return -(-ch // 4) * 4


def _pad_edges(idx, n_chunks):
    e = idx.shape[0]
    total = NW * n_chunks * CHUNK
    pad = N_NODES + (jnp.arange(total - e, dtype=jnp.int32) % (NPAD - N_NODES))
    return jnp.concatenate([idx.astype(jnp.int32), pad]).reshape(NW, n_chunks, CHUNK)


def _sc_mesh():
    return plsc.VectorSubcoreMesh(core_axis_name="c", subcore_axis_name="s",
                                  num_cores=NC, num_subcores=NS)


def _make_deg_kernel(ch_r, ch_k):
    out_t = (jax.ShapeDtypeStruct((NC, NPAD, 128), jnp.float32),
             jax.ShapeDtypeStruct((NC, NPAD, 128), jnp.float32))

    @functools.partial(
        pl.kernel,
        out_type=out_t,
        mesh=_sc_mesh(),
        compiler_params=pltpu.CompilerParams(use_tc_tiling_on_sc=False),
        scratch_types=[
            pltpu.VMEM((ch_r, CHUNK), jnp.int32),
            pltpu.VMEM((ch_k, CHUNK), jnp.int32),
            pltpu.VMEM((CHUNK, 16), jnp.float32),
            pltpu.VMEM_SHARED((NPAD, 16), jnp.float32),
            pltpu.VMEM_SHARED((NPAD, 16), jnp.float32),
        ],
    )
    def deg_kernel(dstr_hbm, dstk_hbm, ones_hbm, zeros_hbm, outr_hbm, outk_hbm,
                   dstr_v, dstk_v, ones_v, acc_r, acc_k):
        c = lax.axis_index("c")
        s = lax.axis_index("s")
        w = s * NC + c
        r0 = s * ROWS_PER_TILE
        pltpu.sync_copy(zeros_hbm.at[pl.ds(r0, ROWS_PER_TILE)],
                        acc_r.at[pl.ds(r0, ROWS_PER_TILE)])
        pltpu.sync_copy(zeros_hbm.at[pl.ds(r0, ROWS_PER_TILE)],
                        acc_k.at[pl.ds(r0, ROWS_PER_TILE)])
        pltpu.sync_copy(dstr_hbm.at[w], dstr_v)
        pltpu.sync_copy(dstk_hbm.at[w], dstk_v)
        pltpu.sync_copy(ones_hbm, ones_v)
        plsc.subcore_barrier()

        def body_r(j, carry):
            pltpu.sync_copy(ones_v, acc_r.at[dstr_v.at[j]], add=True)
            return carry

        lax.fori_loop(0, ch_r, body_r, 0)

        def body_k(j, carry):
            pltpu.sync_copy(ones_v, acc_k.at[dstk_v.at[j]], add=True)
            return carry

        lax.fori_loop(0, ch_k, body_k, 0)
        plsc.subcore_barrier()
        pltpu.sync_copy(acc_r.at[pl.ds(r0, ROWS_PER_TILE)],
                        outr_hbm.at[c].at[pl.ds(r0, ROWS_PER_TILE), pl.ds(0, 16)])
        pltpu.sync_copy(acc_k.at[pl.ds(r0, ROWS_PER_TILE)],
                        outk_hbm.at[c].at[pl.ds(r0, ROWS_PER_TILE), pl.ds(0, 16)])

    return deg_kernel


def _make_agg_kernel(feat, ch_r, ch_k):
    out_t = (jax.ShapeDtypeStruct((NC, NPAD, 128), jnp.float32),
             jax.ShapeDtypeStruct((NC, NPAD, 128), jnp.float32))

    @functools.partial(
        pl.kernel,
        out_type=out_t,
        mesh=_sc_mesh(),
        compiler_params=pltpu.CompilerParams(use_tc_tiling_on_sc=False),
        scratch_types=[
            pltpu.VMEM((ch_r, CHUNK), jnp.int32),
            pltpu.VMEM((ch_r, CHUNK), jnp.int32),
            pltpu.VMEM((ch_k, CHUNK), jnp.int32),
            pltpu.VMEM((ch_k, CHUNK), jnp.int32),
            [pltpu.VMEM((CHUNK, feat), jnp.float32) for _ in range(4)],
            pltpu.VMEM_SHARED((NPAD, feat), jnp.float32),
            [pltpu.SemaphoreType.DMA for _ in range(4)],
            [pltpu.SemaphoreType.DMA for _ in range(4)],
        ],
    )
    def agg_kernel(table_r, table_k, srcr_hbm, dstr_hbm, srck_hbm, dstk_hbm,
                   zeros_hbm, outr_hbm, outk_hbm,
                   srcr_v, dstr_v, srck_v, dstk_v, bufs, acc,
                   sem_g, sem_s):
        c = lax.axis_index("c")
        s = lax.axis_index("s")
        w = s * NC + c
        r0 = s * ROWS_PER_TILE
        pltpu.sync_copy(zeros_hbm.at[pl.ds(r0, ROWS_PER_TILE)],
                        acc.at[pl.ds(r0, ROWS_PER_TILE)])
        pltpu.sync_copy(srcr_hbm.at[w], srcr_v)
        pltpu.sync_copy(dstr_hbm.at[w], dstr_v)
        pltpu.sync_copy(srck_hbm.at[w], srck_v)
        pltpu.sync_copy(dstk_hbm.at[w], dstk_v)
        plsc.subcore_barrier()

        def pipelined(src_v, dst_v, table, n_chunks):
            pltpu.async_copy(table.at[src_v.at[0]], bufs[0], sem_g[0])
            pltpu.async_copy(table.at[src_v.at[1]], bufs[1], sem_g[1])

            def body(g, carry):
                for b in range(4):
                    j = 4 * g + b
                    c2 = (b + 2) % 4
                    pltpu.make_async_copy(table.at[src_v.at[j]], bufs[b],
                                          sem_g[b]).wait()
                    pltpu.async_copy(bufs[b], acc.at[dst_v.at[j]], sem_s[b],
                                     add=True)

                    @pl.when(j >= 2)
                    def _():
                        pltpu.make_async_copy(
                            bufs[c2], acc.at[dst_v.at[j - 2]], sem_s[c2]
                        ).wait()

                    @pl.when(j + 2 < n_chunks)
                    def _():
                        pltpu.async_copy(table.at[src_v.at[j + 2]], bufs[c2],
                                         sem_g[c2])
                return carry

            lax.fori_loop(0, n_chunks // 4, body, 0)
            pltpu.make_async_copy(bufs[2], acc.at[dst_v.at[n_chunks - 2]],
                                  sem_s[2]).wait()
            pltpu.make_async_copy(bufs[3], acc.at[dst_v.at[n_chunks - 1]],
                                  sem_s[3]).wait()

        def flush(out_hbm):
            plsc.subcore_barrier()
            pltpu.sync_copy(acc.at[pl.ds(r0, ROWS_PER_TILE)],
                            out_hbm.at[c].at[pl.ds(r0, ROWS_PER_TILE),
                                             pl.ds(0, feat)])
            pltpu.sync_copy(zeros_hbm.at[pl.ds(r0, ROWS_PER_TILE)],
                            acc.at[pl.ds(r0, ROWS_PER_TILE)])
            plsc.subcore_barrier()

        pipelined(srcr_v, dstr_v, table_r, ch_r)
        flush(outr_hbm)
        pipelined(srck_v, dstk_v, table_k, ch_k)
        plsc.subcore_barrier()
        pltpu.sync_copy(acc.at[pl.ds(r0, ROWS_PER_TILE)],
                        outk_hbm.at[c].at[pl.ds(r0, ROWS_PER_TILE),
                                          pl.ds(0, feat)])

    return agg_kernel


def _tc0(x_pad, W1):
    grid = NPAD // BLK

    def body(x_ref, w_ref, h1_ref):
        h1_ref[...] = jnp.dot(x_ref[...], w_ref[...],
                              preferred_element_type=jnp.float32)

    return pl.pallas_call(
        body,
        grid=(grid,),
        in_specs=[
            pl.BlockSpec((BLK, N_FEAT), lambda i: (i, 0)),
            pl.BlockSpec((N_FEAT, N_HID), lambda i: (0, 0)),
        ],
        out_specs=pl.BlockSpec((BLK, N_HID), lambda i: (i, 0)),
        out_shape=jax.ShapeDtypeStruct((NPAD, N_HID), jnp.float32),
    )(x_pad, W1)


def _tc1(h1, degp_r, degp_k):
    grid = NPAD // BLK

    def body(h1_ref, dr_ref, dk_ref,
             hsr_ref, hsk_ref, dvr_ref, dvk_ref):
        h1 = h1_ref[...]
        deg_r = dr_ref[0, :, 0:1] + dr_ref[1, :, 0:1] + 1.0
        deg_k = dk_ref[0, :, 0:1] + dk_ref[1, :, 0:1] + 1.0
        dinv_r = lax.rsqrt(deg_r)
        dinv_k = lax.rsqrt(deg_k)
        hsr_ref[...] = h1 * dinv_r
        hsk_ref[...] = h1 * dinv_k
        dvr_ref[...] = dinv_r
        dvk_ref[...] = dinv_k

    return pl.pallas_call(
        body,
        grid=(grid,),
        in_specs=[
            pl.BlockSpec((BLK, N_HID), lambda i: (i, 0)),
            pl.BlockSpec((NC, BLK, 128), lambda i: (0, i, 0)),
            pl.BlockSpec((NC, BLK, 128), lambda i: (0, i, 0)),
        ],
        out_specs=[
            pl.BlockSpec((BLK, N_HID), lambda i: (i, 0)),
            pl.BlockSpec((BLK, N_HID), lambda i: (i, 0)),
            pl.BlockSpec((BLK, 1), lambda i: (i, 0)),
            pl.BlockSpec((BLK, 1), lambda i: (i, 0)),
        ],
        out_shape=[
            jax.ShapeDtypeStruct((NPAD, N_HID), jnp.float32),
            jax.ShapeDtypeStruct((NPAD, N_HID), jnp.float32),
            jax.ShapeDtypeStruct((NPAD, 1), jnp.float32),
            jax.ShapeDtypeStruct((NPAD, 1), jnp.float32),
        ],
    )(h1, degp_r, degp_k)


def _tc2(aggp_r, aggp_k, h1, dinv_r, dinv_k, b1, W2):
    grid = NPAD // BLK

    def body(ar_ref, ak_ref, h1_ref, dvr_ref, dvk_ref, b1_ref, w2_ref,
             h2_ref, hsr_ref, hsk_ref):
        dvr = dvr_ref[...]
        dvk = dvk_ref[...]
        h1 = h1_ref[...]
        b1 = b1_ref[...]
        agg_r = ar_ref[0, :, :N_HID] + ar_ref[1, :, :N_HID]
        agg_k = ak_ref[0, :, :N_HID] + ak_ref[1, :, :N_HID]
        conv_r = dvr * agg_r + (dvr * dvr) * h1 + b1
        conv_k = dvk * agg_k + (dvk * dvk) * h1 + b1
        r1 = jax.nn.relu(jnp.concatenate([conv_r, conv_k], axis=1))
        h2 = jnp.dot(r1, w2_ref[...], preferred_element_type=jnp.float32)
        h2_ref[...] = h2
        hsr_ref[...] = h2 * dvr
        hsk_ref[...] = h2 * dvk

    return pl.pallas_call(
        body,
        grid=(grid,),
        in_specs=[
            pl.BlockSpec((NC, BLK, 128), lambda i: (0, i, 0)),
            pl.BlockSpec((NC, BLK, 128), lambda i: (0, i, 0)),
            pl.BlockSpec((BLK, N_HID), lambda i: (i, 0)),
            pl.BlockSpec((BLK, 1), lambda i: (i, 0)),
            pl.BlockSpec((BLK, 1), lambda i: (i, 0)),
            pl.BlockSpec((1, N_HID), lambda i: (0, 0)),
            pl.BlockSpec((2 * N_HID, N_CLS), lambda i: (0, 0)),
        ],
        out_specs=[
            pl.BlockSpec((BLK, N_CLS), lambda i: (i, 0)),
            pl.BlockSpec((BLK, N_CLS), lambda i: (i, 0)),
            pl.BlockSpec((BLK, N_CLS), lambda i: (i, 0)),
        ],
        out_shape=[
            jax.ShapeDtypeStruct((NPAD, N_CLS), jnp.float32),
            jax.ShapeDtypeStruct((NPAD, N_CLS), jnp.float32),
            jax.ShapeDtypeStruct((NPAD, N_CLS), jnp.float32),
        ],
    )(aggp_r, aggp_k, h1, dinv_r, dinv_k, b1, W2)


def _tc3(aggp_r, aggp_k, h2, dinv_r, dinv_k, b2, Wl, bl):
    grid = NPAD // BLK

    def body(ar_ref, ak_ref, h2_ref, dvr_ref, dvk_ref, b2_ref, wl_ref, bl_ref,
             out_ref):
        dvr = dvr_ref[...]
        dvk = dvk_ref[...]
        h2 = h2_ref[...]
        b2 = b2_ref[...]
        agg_r = ar_ref[0, :, :N_CLS] + ar_ref[1, :, :N_CLS]
        agg_k = ak_ref[0, :, :N_CLS] + ak_ref[1, :, :N_CLS]
        conv_r = dvr * agg_r + (dvr * dvr) * h2 + b2
        conv_k = dvk * agg_k + (dvk * dvk) * h2 + b2
        r2 = jnp.concatenate([conv_r, conv_k], axis=1)
        final = jnp.dot(r2, wl_ref[...], preferred_element_type=jnp.float32)
        final = final + bl_ref[...]
        m = jnp.max(final, axis=1, keepdims=True)
        lse = jnp.log(jnp.sum(jnp.exp(final - m), axis=1, keepdims=True)) + m
        out_ref[...] = final - lse

    return pl.pallas_call(
        body,
        grid=(grid,),
        in_specs=[
            pl.BlockSpec((NC, BLK, 128), lambda i: (0, i, 0)),
            pl.BlockSpec((NC, BLK, 128), lambda i: (0, i, 0)),
            pl.BlockSpec((BLK, N_CLS), lambda i: (i, 0)),
            pl.BlockSpec((BLK, 1), lambda i: (i, 0)),
            pl.BlockSpec((BLK, 1), lambda i: (i, 0)),
            pl.BlockSpec((1, N_CLS), lambda i: (0, 0)),
            pl.BlockSpec((2 * N_CLS, N_CLS), lambda i: (0, 0)),
            pl.BlockSpec((1, N_CLS), lambda i: (0, 0)),
        ],
        out_specs=pl.BlockSpec((BLK, N_CLS), lambda i: (i, 0)),
        out_shape=jax.ShapeDtypeStruct((NPAD, N_CLS), jnp.float32),
    )(aggp_r, aggp_k, h2, dinv_r, dinv_k, b2, Wl, bl)


def kernel(x, edge_index, edge_index_knn, W1, b1, W2, b2, Wl, bl):
    e_r = edge_index.shape[1]
    e_k = edge_index_knn.shape[1]
    ch_r = _n_chunks(e_r)
    ch_k = _n_chunks(e_k)

    src_r = _pad_edges(edge_index[0], ch_r)
    dst_r = _pad_edges(edge_index[1], ch_r)
    src_k = _pad_edges(edge_index_knn[0], ch_k)
    dst_k = _pad_edges(edge_index_knn[1], ch_k)

    x_pad = jnp.zeros((NPAD, N_FEAT), jnp.float32).at[:N_NODES].set(x)
    ones16 = jnp.ones((CHUNK, 16), jnp.float32)
    zeros16 = jnp.zeros((NPAD, 16), jnp.float32)
    zeros_h = jnp.zeros((NPAD, N_HID), jnp.float32)
    zeros_c = jnp.zeros((NPAD, N_CLS), jnp.float32)
    b1r = b1.reshape(1, N_HID)
    b2r = b2.reshape(1, N_CLS)
    blr = bl.reshape(1, N_CLS)

    h1 = _tc0(x_pad, W1)
    degp_r, degp_k = _make_deg_kernel(ch_r, ch_k)(dst_r, dst_k, ones16, zeros16)

    hs1_r, hs1_k, dinv_r, dinv_k = _tc1(h1, degp_r, degp_k)

    agg1 = _make_agg_kernel(N_HID, ch_r, ch_k)
    aggp1_r, aggp1_k = agg1(hs1_r, hs1_k, src_r, dst_r, src_k, dst_k, zeros_h)

    h2, hs2_r, hs2_k = _tc2(aggp1_r, aggp1_k, h1, dinv_r, dinv_k, b1r, W2)

    agg2 = _make_agg_kernel(N_CLS, ch_r, ch_k)
    aggp2_r, aggp2_k = agg2(hs2_r, hs2_k, src_r, dst_r, src_k, dst_k, zeros_c)

    out = _tc3(aggp2_r, aggp2_k, h2, dinv_r, dinv_k, b2r, Wl, blr)
    return out[:N_NODES]
